# Initial kernel scaffold; baseline (speedup 1.0000x reference)
import jax
import jax.numpy as jnp
from jax.experimental import pallas as pl


def kernel(node_features, edge_index, edge_attr, coords, We1, be1, We2, be2, Wn1, bn1, Wn2, bn2, Wc1, bc1, Wc2, bc2):
    raise NotImplementedError("write your pallas kernel here")


# speedup vs baseline: 2.7942x; 2.7942x over previous
import functools

import jax
import jax.numpy as jnp
from jax import lax
from jax.experimental import pallas as pl
from jax.experimental.pallas import tpu as pltpu
from jax.experimental.pallas import tpu_sc as plsc

_NC = 2
_NS = 16
_NW = _NC * _NS
_CG = 80


def _prep_body(nf_ref, we1_ref, we2_ref, wc1_ref, bc1_ref, be2_ref,
               a_ref, b_ref, w2c_ref, bc1e_ref):
    d = we2_ref.shape[0]
    nf = nf_ref[...]
    we1 = we1_ref[...]
    wc1 = wc1_ref[...]
    a_ref[...] = jnp.dot(nf, we1[0:d, :], preferred_element_type=jnp.float32)
    b_ref[...] = jnp.dot(nf, we1[d:2 * d, :], preferred_element_type=jnp.float32)
    w2c_ref[...] = jnp.dot(we2_ref[...], wc1, preferred_element_type=jnp.float32)
    bc1e_ref[...] = bc1_ref[...] + jnp.dot(
        be2_ref[...], wc1, preferred_element_type=jnp.float32)


def _edge_body(ar_ref, bc_ref, ea_ref, xr_ref, yr_ref, zr_ref,
               xc_ref, yc_ref, zc_ref,
               we1_ref, be1_ref, w2c_ref, bc1e_ref, wc2_ref, bc2_ref,
               mh_ref, cm4_ref):
    d = w2c_ref.shape[0]
    ed = ea_ref.shape[1]
    w1e = we1_ref[2 * d:2 * d + ed, :]
    pre = (ar_ref[...] + bc_ref[...]
           + jnp.dot(ea_ref[...], w1e, preferred_element_type=jnp.float32)
           + be1_ref[...])
    h = pre * jax.nn.sigmoid(pre)
    chp = jnp.dot(h, w2c_ref[...], preferred_element_type=jnp.float32) + bc1e_ref[...]
    ch = chp * jax.nn.sigmoid(chp)
    mh_ref[...] = h
    s = jnp.dot(ch, wc2_ref[...], preferred_element_type=jnp.float32) + bc2_ref[...]
    cdx = xr_ref[0] - xc_ref[0]
    cdy = yr_ref[0] - yc_ref[0]
    cdz = zr_ref[0] - zc_ref[0]
    cdstack = jnp.concatenate([cdx, cdy, cdz, jnp.zeros_like(cdx)], axis=0)
    rr = lax.broadcasted_iota(jnp.int32, (4, d), 0)
    cc = lax.broadcasted_iota(jnp.int32, (4, d), 1)
    proj = (rr == cc).astype(jnp.float32)
    cd4 = lax.dot_general(cdstack, proj,
                          dimension_numbers=(((0,), (0,)), ((), ())),
                          preferred_element_type=jnp.float32)
    dist = jnp.sqrt(jnp.sum(cd4 * cd4, axis=1, keepdims=True)) + 1e-8
    f = s / dist
    lane = lax.broadcasted_iota(jnp.int32, cd4.shape, 1)
    cm4_ref[...] = jnp.where(lane == 3, 1.0, f * cd4)


def _post_body(nf_ref, hacc_ref, cacc_ref, cpad_ref,
               we2_ref, be2_ref, wn1_ref, bn1_ref, wn2_ref, bn2_ref,
               out1_ref, out2_ref):
    d = we2_ref.shape[0]
    hs = hacc_ref[...]
    cacc = cacc_ref[...]
    deg = cacc[:, 3:4]
    nm = jnp.dot(hs, we2_ref[...], preferred_element_type=jnp.float32) + deg * be2_ref[...]
    wn1 = wn1_ref[...]
    pre = (jnp.dot(nf_ref[...], wn1[0:d, :], preferred_element_type=jnp.float32)
           + jnp.dot(nm, wn1[d:2 * d, :], preferred_element_type=jnp.float32)
           + bn1_ref[...])
    h2 = pre * jax.nn.sigmoid(pre)
    out1_ref[...] = jnp.dot(h2, wn2_ref[...], preferred_element_type=jnp.float32) + bn2_ref[...]
    out2_ref[...] = cpad_ref[...] + cacc[:, 0:4]


def _make_gather(n, e, d):
    ew = e // _NW
    nch = ew // _CG
    mesh = plsc.VectorSubcoreMesh(core_axis_name="c", subcore_axis_name="s")
    f32 = jnp.float32

    @functools.partial(
        pl.kernel,
        out_type=[jax.ShapeDtypeStruct((e, d), f32),
                  jax.ShapeDtypeStruct((e, d), f32)]
        + [jax.ShapeDtypeStruct((e,), f32)] * 6,
        mesh=mesh,
        scratch_types=[
            pltpu.VMEM((_CG,), jnp.int32),
            pltpu.VMEM((_CG,), jnp.int32),
            pltpu.VMEM((_CG, d), f32),
            pltpu.VMEM((_CG, d), f32),
            pltpu.VMEM((_CG,), f32),
            pltpu.VMEM((_CG,), f32),
            pltpu.VMEM((_CG,), f32),
            pltpu.VMEM((_CG,), f32),
            pltpu.VMEM((_CG,), f32),
            pltpu.VMEM((_CG,), f32),
            pltpu.SemaphoreType.DMA,
        ],
    )
    def gather_k(a_hbm, b_hbm, x_hbm, y_hbm, z_hbm, row_hbm, col_hbm,
                 ar_hbm, bc_hbm, xr_hbm, yr_hbm, zr_hbm, xc_hbm, yc_hbm, zc_hbm,
                 idxr_v, idxc_v, abuf, bbuf, xrb, yrb, zrb, xcb, ycb, zcb, sem):
        wid = lax.axis_index("s") * _NC + lax.axis_index("c")
        base = wid * ew

        def body(j, carry):
            off = base + j * _CG
            sl = pl.ds(off, _CG)
            pltpu.sync_copy(row_hbm.at[sl], idxr_v)
            pltpu.sync_copy(col_hbm.at[sl], idxc_v)
            cps = [
                pltpu.async_copy(a_hbm.at[idxr_v], abuf, sem),
                pltpu.async_copy(b_hbm.at[idxc_v], bbuf, sem),
                pltpu.async_copy(x_hbm.at[idxr_v], xrb, sem),
                pltpu.async_copy(y_hbm.at[idxr_v], yrb, sem),
                pltpu.async_copy(z_hbm.at[idxr_v], zrb, sem),
                pltpu.async_copy(x_hbm.at[idxc_v], xcb, sem),
                pltpu.async_copy(y_hbm.at[idxc_v], ycb, sem),
                pltpu.async_copy(z_hbm.at[idxc_v], zcb, sem),
            ]
            for c in cps:
                c.wait()
            pltpu.sync_copy(abuf, ar_hbm.at[sl])
            pltpu.sync_copy(bbuf, bc_hbm.at[sl])
            pltpu.sync_copy(xrb, xr_hbm.at[sl])
            pltpu.sync_copy(yrb, yr_hbm.at[sl])
            pltpu.sync_copy(zrb, zr_hbm.at[sl])
            pltpu.sync_copy(xcb, xc_hbm.at[sl])
            pltpu.sync_copy(ycb, yc_hbm.at[sl])
            pltpu.sync_copy(zcb, zc_hbm.at[sl])
            return carry

        lax.fori_loop(0, nch, body, 0)

    return gather_k


def _make_scatter(n, e, d):
    et = e // _NS
    nch = et // _CG
    nblk = (n + _CG - 1) // _CG
    mesh = plsc.VectorSubcoreMesh(core_axis_name="c", subcore_axis_name="s")
    f32 = jnp.float32

    @functools.partial(
        pl.kernel,
        out_type=[jax.ShapeDtypeStruct((n, d), f32),
                  jax.ShapeDtypeStruct((n, d), f32)],
        mesh=mesh,
        scratch_types=[
            pltpu.VMEM((_CG,), jnp.int32),
            pltpu.VMEM((_CG, d), f32),
            pltpu.VMEM((_CG, d), f32),
            pltpu.VMEM_SHARED((n, d), f32),
        ],
    )
    def scatter_k(mh_hbm, cm4_hbm, row_hbm, zh_hbm,
                  hout_hbm, cout_hbm,
                  idx_v, mhbuf, hbb, acc):
        cid = lax.axis_index("c")
        sid = lax.axis_index("s")

        def zinit(j, carry):
            b = j * _NS + sid

            @pl.when(b < nblk)
            def _():
                rs = pl.ds(b * _CG, _CG)
                pltpu.sync_copy(zh_hbm.at[rs], hbb)
                pltpu.sync_copy(hbb, acc.at[rs])

            return carry

        lax.fori_loop(0, (nblk + _NS - 1) // _NS, zinit, 0)
        plsc.subcore_barrier()

        ebase = sid * et

        def mk_body(src_hbm):
            def body(j, carry):
                off = ebase + j * _CG
                sl = pl.ds(off, _CG)
                pltpu.sync_copy(row_hbm.at[sl], idx_v)
                pltpu.sync_copy(src_hbm.at[sl], mhbuf)
                pltpu.sync_copy(mhbuf, acc.at[idx_v], add=True)
                return carry
            return body

        @pl.when(cid == 0)
        def _():
            lax.fori_loop(0, nch, mk_body(mh_hbm), 0)

        @pl.when(cid == 1)
        def _():
            lax.fori_loop(0, nch, mk_body(cm4_hbm), 0)

        plsc.subcore_barrier()

        def mk_drain(dst_hbm):
            def drain(j, carry):
                b = j * _NS + sid

                @pl.when(b < nblk)
                def _():
                    rs = pl.ds(b * _CG, _CG)
                    pltpu.sync_copy(acc.at[rs], hbb)
                    pltpu.sync_copy(hbb, dst_hbm.at[rs])

                return carry
            return drain

        @pl.when(cid == 0)
        def _():
            lax.fori_loop(0, (nblk + _NS - 1) // _NS, mk_drain(hout_hbm), 0)

        @pl.when(cid == 1)
        def _():
            lax.fori_loop(0, (nblk + _NS - 1) // _NS, mk_drain(cout_hbm), 0)

    return scatter_k


def kernel(node_features, edge_index, edge_attr, coords,
           We1, be1, We2, be2, Wn1, bn1, Wn2, bn2, Wc1, bc1, Wc2, bc2):
    n, d = node_features.shape
    e = edge_index.shape[1]
    ed = edge_attr.shape[1]

    row = edge_index[0]
    col = edge_index[1]
    ct = coords.T
    cx, cy, cz = ct[0], ct[1], ct[2]
    cpad = jnp.pad(coords, ((0, 0), (0, 1)))
    be1r = be1.reshape(1, d)
    be2r = be2.reshape(1, d)
    bn1r = bn1.reshape(1, d)
    bn2r = bn2.reshape(1, d)
    bc1r = bc1.reshape(1, d)
    bc2r = bc2.reshape(1, 1)

    f32 = jnp.float32
    a, b, w2c, bc1e = pl.pallas_call(
        _prep_body,
        out_shape=[jax.ShapeDtypeStruct((n, d), f32),
                   jax.ShapeDtypeStruct((n, d), f32),
                   jax.ShapeDtypeStruct((d, d), f32),
                   jax.ShapeDtypeStruct((1, d), f32)],
    )(node_features, We1, We2, Wc1, bc1r, be2r)

    ar, bc, xr, yr, zr, xc, yc, zc = _make_gather(n, e, d)(
        a, b, cx, cy, cz, row, col)

    be = 512
    grid_e = e // be
    lane3 = (grid_e, 1, be)
    r3 = lambda v: v.reshape(lane3)
    full = lambda shape: pl.BlockSpec(shape, lambda i: (0,) * len(shape))
    lane_spec = pl.BlockSpec((1, 1, be), lambda i: (i, 0, 0))
    mh, cm4 = pl.pallas_call(
        _edge_body,
        grid=(grid_e,),
        in_specs=[
            pl.BlockSpec((be, d), lambda i: (i, 0)),
            pl.BlockSpec((be, d), lambda i: (i, 0)),
            pl.BlockSpec((be, ed), lambda i: (i, 0)),
            lane_spec, lane_spec, lane_spec, lane_spec, lane_spec, lane_spec,
            full((2 * d + ed, d)),
            full((1, d)),
            full((d, d)),
            full((1, d)),
            full((d, 1)),
            full((1, 1)),
        ],
        out_specs=[
            pl.BlockSpec((be, d), lambda i: (i, 0)),
            pl.BlockSpec((be, d), lambda i: (i, 0)),
        ],
        out_shape=[jax.ShapeDtypeStruct((e, d), f32),
                   jax.ShapeDtypeStruct((e, d), f32)],
    )(ar, bc, edge_attr, r3(xr), r3(yr), r3(zr), r3(xc), r3(yc), r3(zc),
      We1, be1r, w2c, bc1e, Wc2, bc2r)

    zh = jnp.zeros((n, d), f32)
    hacc, cacc = _make_scatter(n, e, d)(mh, cm4, row, zh)

    bn = 1000
    grid_n = n // bn
    nf_new, cnew = pl.pallas_call(
        _post_body,
        grid=(grid_n,),
        in_specs=[
            pl.BlockSpec((bn, d), lambda i: (i, 0)),
            pl.BlockSpec((bn, d), lambda i: (i, 0)),
            pl.BlockSpec((bn, d), lambda i: (i, 0)),
            pl.BlockSpec((bn, 4), lambda i: (i, 0)),
            full((d, d)),
            full((1, d)),
            full((2 * d, d)),
            full((1, d)),
            full((d, d)),
            full((1, d)),
        ],
        out_specs=[
            pl.BlockSpec((bn, d), lambda i: (i, 0)),
            pl.BlockSpec((bn, 4), lambda i: (i, 0)),
        ],
        out_shape=[jax.ShapeDtypeStruct((n, d), f32),
                   jax.ShapeDtypeStruct((n, 4), f32)],
    )(node_features, hacc, cacc, cpad, We2, be2r, Wn1, bn1r, Wn2, bn2r)

    return nf_new, cnew[:, :coords.shape[1]]


# speedup vs baseline: 3.3650x; 1.2043x over previous
import functools

import jax
import jax.numpy as jnp
from jax import lax
from jax.experimental import pallas as pl
from jax.experimental.pallas import tpu as pltpu
from jax.experimental.pallas import tpu_sc as plsc

_NC = 2
_NS = 16
_NW = _NC * _NS
_CG = 80


def _prep_body(nf_ref, we1_ref, we2_ref, wc1_ref, bc1_ref, be2_ref,
               a_ref, b_ref, w2c_ref, bc1e_ref):
    d = we2_ref.shape[0]
    nf = nf_ref[...]
    we1 = we1_ref[...]
    wc1 = wc1_ref[...]
    a_ref[...] = jnp.dot(nf, we1[0:d, :], preferred_element_type=jnp.float32)
    b_ref[...] = jnp.dot(nf, we1[d:2 * d, :], preferred_element_type=jnp.float32)
    w2c_ref[...] = jnp.dot(we2_ref[...], wc1, preferred_element_type=jnp.float32)
    bc1e_ref[...] = bc1_ref[...] + jnp.dot(
        be2_ref[...], wc1, preferred_element_type=jnp.float32)


def _edge_body(ar_ref, bc_ref, ea_ref, xr_ref, yr_ref, zr_ref,
               xc_ref, yc_ref, zc_ref,
               we1_ref, be1_ref, w2c_ref, bc1e_ref, wc2_ref, bc2_ref,
               mh_ref, cm4_ref):
    d = w2c_ref.shape[0]
    ed = ea_ref.shape[1]
    w1e = we1_ref[2 * d:2 * d + ed, :]
    pre = (ar_ref[...] + bc_ref[...]
           + jnp.dot(ea_ref[...], w1e, preferred_element_type=jnp.float32)
           + be1_ref[...])
    h = pre * jax.nn.sigmoid(pre)
    chp = jnp.dot(h, w2c_ref[...], preferred_element_type=jnp.float32) + bc1e_ref[...]
    ch = chp * jax.nn.sigmoid(chp)
    mh_ref[...] = h
    s = jnp.dot(ch, wc2_ref[...], preferred_element_type=jnp.float32) + bc2_ref[...]
    cdx = xr_ref[0] - xc_ref[0]
    cdy = yr_ref[0] - yc_ref[0]
    cdz = zr_ref[0] - zc_ref[0]
    cdstack = jnp.concatenate([cdx, cdy, cdz, jnp.zeros_like(cdx)], axis=0)
    rr = lax.broadcasted_iota(jnp.int32, (4, d), 0)
    cc = lax.broadcasted_iota(jnp.int32, (4, d), 1)
    proj = (rr == cc).astype(jnp.float32)
    cd4 = lax.dot_general(cdstack, proj,
                          dimension_numbers=(((0,), (0,)), ((), ())),
                          preferred_element_type=jnp.float32)
    dist = jnp.sqrt(jnp.sum(cd4 * cd4, axis=1, keepdims=True)) + 1e-8
    f = s / dist
    lane = lax.broadcasted_iota(jnp.int32, cd4.shape, 1)
    cm4_ref[...] = jnp.where(lane == 3, 1.0, f * cd4)


def _post_body(nf_ref, hacc_ref, cacc_ref, cpad_ref,
               we2_ref, be2_ref, wn1_ref, bn1_ref, wn2_ref, bn2_ref,
               out1_ref, out2_ref):
    d = we2_ref.shape[0]
    hs = hacc_ref[...]
    cacc = cacc_ref[...]
    deg = cacc[:, 3:4]
    nm = jnp.dot(hs, we2_ref[...], preferred_element_type=jnp.float32) + deg * be2_ref[...]
    wn1 = wn1_ref[...]
    pre = (jnp.dot(nf_ref[...], wn1[0:d, :], preferred_element_type=jnp.float32)
           + jnp.dot(nm, wn1[d:2 * d, :], preferred_element_type=jnp.float32)
           + bn1_ref[...])
    h2 = pre * jax.nn.sigmoid(pre)
    out1_ref[...] = jnp.dot(h2, wn2_ref[...], preferred_element_type=jnp.float32) + bn2_ref[...]
    out2_ref[...] = cpad_ref[...] + cacc[:, 0:4]


def _make_gather(n, e, d):
    ew = e // _NW
    nch = ew // _CG
    mesh = plsc.VectorSubcoreMesh(core_axis_name="c", subcore_axis_name="s")
    f32 = jnp.float32

    @functools.partial(
        pl.kernel,
        out_type=[jax.ShapeDtypeStruct((e, d), f32),
                  jax.ShapeDtypeStruct((e, d), f32)]
        + [jax.ShapeDtypeStruct((e,), f32)] * 6,
        mesh=mesh,
        scratch_types=[
            pltpu.VMEM((2, _CG), jnp.int32),
            pltpu.VMEM((2, _CG), jnp.int32),
            pltpu.VMEM((2, _CG, d), f32),
            pltpu.VMEM((2, _CG, d), f32),
            pltpu.VMEM((2, _CG), f32),
            pltpu.VMEM((2, _CG), f32),
            pltpu.VMEM((2, _CG), f32),
            pltpu.VMEM((2, _CG), f32),
            pltpu.VMEM((2, _CG), f32),
            pltpu.VMEM((2, _CG), f32),
            pltpu.SemaphoreType.DMA,
            pltpu.SemaphoreType.DMA,
        ],
    )
    def gather_k(a_hbm, b_hbm, x_hbm, y_hbm, z_hbm, row_hbm, col_hbm,
                 ar_hbm, bc_hbm, xr_hbm, yr_hbm, zr_hbm, xc_hbm, yc_hbm, zc_hbm,
                 idxr_v, idxc_v, abuf, bbuf, xrb, yrb, zrb, xcb, ycb, zcb,
                 semg, semw):
        wid = lax.axis_index("s") * _NC + lax.axis_index("c")
        base = wid * ew
        smalls = [xrb, yrb, zrb, xcb, ycb, zcb]
        souts = [xr_hbm, yr_hbm, zr_hbm, xc_hbm, yc_hbm, zc_hbm]

        def fire(j, p):
            sl = pl.ds(base + j * _CG, _CG)
            pltpu.sync_copy(row_hbm.at[sl], idxr_v.at[p])
            pltpu.sync_copy(col_hbm.at[sl], idxc_v.at[p])
            pltpu.async_copy(a_hbm.at[idxr_v.at[p]], abuf.at[p], semg)
            pltpu.async_copy(b_hbm.at[idxc_v.at[p]], bbuf.at[p], semg)
            for k in range(3):
                t = [x_hbm, y_hbm, z_hbm][k]
                pltpu.async_copy(t.at[idxr_v.at[p]], smalls[k].at[p], semg)
                pltpu.async_copy(t.at[idxc_v.at[p]], smalls[3 + k].at[p], semg)

        def wait_gathers(p):
            pltpu.make_async_copy(a_hbm.at[idxr_v.at[p]], abuf.at[p], semg).wait()
            pltpu.make_async_copy(b_hbm.at[idxc_v.at[p]], bbuf.at[p], semg).wait()
            for k in range(6):
                pltpu.make_async_copy(
                    x_hbm.at[idxr_v.at[p]], smalls[k].at[p], semg).wait()

        def fire_wb(j, p):
            sl = pl.ds(base + j * _CG, _CG)
            pltpu.async_copy(abuf.at[p], ar_hbm.at[sl], semw)
            pltpu.async_copy(bbuf.at[p], bc_hbm.at[sl], semw)
            for k in range(6):
                pltpu.async_copy(smalls[k].at[p], souts[k].at[sl], semw)

        def wait_wb(p):
            sl = pl.ds(base, _CG)
            pltpu.make_async_copy(abuf.at[p], ar_hbm.at[sl], semw).wait()
            pltpu.make_async_copy(bbuf.at[p], bc_hbm.at[sl], semw).wait()
            for k in range(6):
                pltpu.make_async_copy(smalls[k].at[p], souts[k].at[sl], semw).wait()

        fire(0, 0)
        wait_gathers(0)
        fire_wb(0, 0)

        def pair(j2, carry):
            for p, coff in ((1, 1), (0, 2)):
                c = 2 * j2 + coff
                fire(c, p)
                wait_wb(1 - p)
                wait_gathers(p)
                fire_wb(c, p)
            return carry

        lax.fori_loop(0, (nch - 1) // 2, pair, 0)
        wait_wb(0)

    return gather_k


def _make_scatter(n, e, d):
    et = e // _NS
    nch = et // _CG
    nblk = (n + _CG - 1) // _CG
    mesh = plsc.VectorSubcoreMesh(core_axis_name="c", subcore_axis_name="s")
    f32 = jnp.float32

    @functools.partial(
        pl.kernel,
        out_type=[jax.ShapeDtypeStruct((n, d), f32),
                  jax.ShapeDtypeStruct((n, d), f32)],
        mesh=mesh,
        scratch_types=[
            pltpu.VMEM((2, _CG), jnp.int32),
            pltpu.VMEM((2, _CG, d), f32),
            pltpu.VMEM((_CG, d), f32),
            pltpu.VMEM_SHARED((n, d), f32),
            pltpu.SemaphoreType.DMA,
        ],
    )
    def scatter_k(mh_hbm, cm4_hbm, row_hbm, zh_hbm,
                  hout_hbm, cout_hbm,
                  idx_v, mhbuf, hbb, acc, seml):
        cid = lax.axis_index("c")
        sid = lax.axis_index("s")

        def zinit(j, carry):
            b = j * _NS + sid

            @pl.when(b < nblk)
            def _():
                rs = pl.ds(b * _CG, _CG)
                pltpu.sync_copy(zh_hbm.at[rs], hbb)
                pltpu.sync_copy(hbb, acc.at[rs])

            return carry

        lax.fori_loop(0, (nblk + _NS - 1) // _NS, zinit, 0)
        plsc.subcore_barrier()

        ebase = sid * et

        def mk_sweep(src_hbm):
            def fire(j, p):
                sl = pl.ds(ebase + j * _CG, _CG)
                pltpu.async_copy(row_hbm.at[sl], idx_v.at[p], seml)
                pltpu.async_copy(src_hbm.at[sl], mhbuf.at[p], seml)

            def wait_loads(p):
                sl = pl.ds(ebase, _CG)
                pltpu.make_async_copy(row_hbm.at[sl], idx_v.at[p], seml).wait()
                pltpu.make_async_copy(src_hbm.at[sl], mhbuf.at[p], seml).wait()

            def sweep():
                fire(0, 0)

                def pair(j2, carry):
                    for p, coff in ((0, 0), (1, 1)):
                        c = 2 * j2 + coff
                        wait_loads(p)

                        @pl.when(c + 1 < nch)
                        def _():
                            fire(c + 1, 1 - p)

                        pltpu.sync_copy(mhbuf.at[p], acc.at[idx_v.at[p]],
                                        add=True)
                    return carry

                lax.fori_loop(0, (nch + 1) // 2, pair, 0)
            return sweep

        @pl.when(cid == 0)
        def _():
            mk_sweep(mh_hbm)()

        @pl.when(cid == 1)
        def _():
            mk_sweep(cm4_hbm)()

        plsc.subcore_barrier()

        def mk_drain(dst_hbm):
            def drain(j, carry):
                b = j * _NS + sid

                @pl.when(b < nblk)
                def _():
                    rs = pl.ds(b * _CG, _CG)
                    pltpu.sync_copy(acc.at[rs], hbb)
                    pltpu.sync_copy(hbb, dst_hbm.at[rs])

                return carry
            return drain

        @pl.when(cid == 0)
        def _():
            lax.fori_loop(0, (nblk + _NS - 1) // _NS, mk_drain(hout_hbm), 0)

        @pl.when(cid == 1)
        def _():
            lax.fori_loop(0, (nblk + _NS - 1) // _NS, mk_drain(cout_hbm), 0)

    return scatter_k


def kernel(node_features, edge_index, edge_attr, coords,
           We1, be1, We2, be2, Wn1, bn1, Wn2, bn2, Wc1, bc1, Wc2, bc2):
    n, d = node_features.shape
    e = edge_index.shape[1]
    ed = edge_attr.shape[1]

    row = edge_index[0]
    col = edge_index[1]
    ct = coords.T
    cx, cy, cz = ct[0], ct[1], ct[2]
    cpad = jnp.pad(coords, ((0, 0), (0, 1)))
    be1r = be1.reshape(1, d)
    be2r = be2.reshape(1, d)
    bn1r = bn1.reshape(1, d)
    bn2r = bn2.reshape(1, d)
    bc1r = bc1.reshape(1, d)
    bc2r = bc2.reshape(1, 1)

    f32 = jnp.float32
    a, b, w2c, bc1e = pl.pallas_call(
        _prep_body,
        out_shape=[jax.ShapeDtypeStruct((n, d), f32),
                   jax.ShapeDtypeStruct((n, d), f32),
                   jax.ShapeDtypeStruct((d, d), f32),
                   jax.ShapeDtypeStruct((1, d), f32)],
    )(node_features, We1, We2, Wc1, bc1r, be2r)

    ar, bc, xr, yr, zr, xc, yc, zc = _make_gather(n, e, d)(
        a, b, cx, cy, cz, row, col)

    be = 512
    grid_e = e // be
    lane3 = (grid_e, 1, be)
    r3 = lambda v: v.reshape(lane3)
    full = lambda shape: pl.BlockSpec(shape, lambda i: (0,) * len(shape))
    lane_spec = pl.BlockSpec((1, 1, be), lambda i: (i, 0, 0))
    mh, cm4 = pl.pallas_call(
        _edge_body,
        grid=(grid_e,),
        in_specs=[
            pl.BlockSpec((be, d), lambda i: (i, 0)),
            pl.BlockSpec((be, d), lambda i: (i, 0)),
            pl.BlockSpec((be, ed), lambda i: (i, 0)),
            lane_spec, lane_spec, lane_spec, lane_spec, lane_spec, lane_spec,
            full((2 * d + ed, d)),
            full((1, d)),
            full((d, d)),
            full((1, d)),
            full((d, 1)),
            full((1, 1)),
        ],
        out_specs=[
            pl.BlockSpec((be, d), lambda i: (i, 0)),
            pl.BlockSpec((be, d), lambda i: (i, 0)),
        ],
        out_shape=[jax.ShapeDtypeStruct((e, d), f32),
                   jax.ShapeDtypeStruct((e, d), f32)],
    )(ar, bc, edge_attr, r3(xr), r3(yr), r3(zr), r3(xc), r3(yc), r3(zc),
      We1, be1r, w2c, bc1e, Wc2, bc2r)

    zh = jnp.zeros((n, d), f32)
    hacc, cacc = _make_scatter(n, e, d)(mh, cm4, row, zh)

    bn = 1000
    grid_n = n // bn
    nf_new, cnew = pl.pallas_call(
        _post_body,
        grid=(grid_n,),
        in_specs=[
            pl.BlockSpec((bn, d), lambda i: (i, 0)),
            pl.BlockSpec((bn, d), lambda i: (i, 0)),
            pl.BlockSpec((bn, d), lambda i: (i, 0)),
            pl.BlockSpec((bn, 4), lambda i: (i, 0)),
            full((d, d)),
            full((1, d)),
            full((2 * d, d)),
            full((1, d)),
            full((d, d)),
            full((1, d)),
        ],
        out_specs=[
            pl.BlockSpec((bn, d), lambda i: (i, 0)),
            pl.BlockSpec((bn, 4), lambda i: (i, 0)),
        ],
        out_shape=[jax.ShapeDtypeStruct((n, d), f32),
                   jax.ShapeDtypeStruct((n, 4), f32)],
    )(node_features, hacc, cacc, cpad, We2, be2r, Wn1, bn1r, Wn2, bn2r)

    return nf_new, cnew[:, :coords.shape[1]]


# speedup vs baseline: 3.3705x; 1.0016x over previous
import functools

import jax
import jax.numpy as jnp
from jax import lax
from jax.experimental import pallas as pl
from jax.experimental.pallas import tpu as pltpu
from jax.experimental.pallas import tpu_sc as plsc

_NC = 2
_NS = 16
_NW = _NC * _NS
_CG = 80


def _prep_body(nf_ref, we1_ref, we2_ref, wc1_ref, bc1_ref, be2_ref,
               a_ref, b_ref, w2c_ref, bc1e_ref):
    d = we2_ref.shape[0]
    nf = nf_ref[...]
    we1 = we1_ref[...]
    wc1 = wc1_ref[...]
    a_ref[...] = jnp.dot(nf, we1[0:d, :], preferred_element_type=jnp.float32)
    b_ref[...] = jnp.dot(nf, we1[d:2 * d, :], preferred_element_type=jnp.float32)
    w2c_ref[...] = jnp.dot(we2_ref[...], wc1, preferred_element_type=jnp.float32)
    bc1e_ref[...] = bc1_ref[...] + jnp.dot(
        be2_ref[...], wc1, preferred_element_type=jnp.float32)


def _edge_body(ar_ref, bc_ref, ea_ref, xr_ref, yr_ref, zr_ref,
               xc_ref, yc_ref, zc_ref,
               we1_ref, be1_ref, w2c_ref, bc1e_ref, wc2_ref, bc2_ref,
               mh_ref, cm4_ref):
    d = w2c_ref.shape[0]
    ed = ea_ref.shape[1]
    w1e = we1_ref[2 * d:2 * d + ed, :]
    pre = (ar_ref[...] + bc_ref[...]
           + jnp.dot(ea_ref[...], w1e, preferred_element_type=jnp.float32)
           + be1_ref[...])
    h = pre * jax.nn.sigmoid(pre)
    chp = jnp.dot(h.astype(jnp.bfloat16), w2c_ref[...].astype(jnp.bfloat16),
                  preferred_element_type=jnp.float32) + bc1e_ref[...]
    ch = chp * jax.nn.sigmoid(chp)
    mh_ref[...] = h
    s = jnp.dot(ch, wc2_ref[...], preferred_element_type=jnp.float32) + bc2_ref[...]
    cdx = xr_ref[0] - xc_ref[0]
    cdy = yr_ref[0] - yc_ref[0]
    cdz = zr_ref[0] - zc_ref[0]
    cdstack = jnp.concatenate([cdx, cdy, cdz, jnp.zeros_like(cdx)], axis=0)
    rr = lax.broadcasted_iota(jnp.int32, (4, d), 0)
    cc = lax.broadcasted_iota(jnp.int32, (4, d), 1)
    proj = (rr == cc).astype(jnp.float32)
    cd4 = lax.dot_general(cdstack, proj,
                          dimension_numbers=(((0,), (0,)), ((), ())),
                          preferred_element_type=jnp.float32)
    dist = jnp.sqrt(jnp.sum(cd4 * cd4, axis=1, keepdims=True)) + 1e-8
    f = s / dist
    lane = lax.broadcasted_iota(jnp.int32, cd4.shape, 1)
    cm4_ref[...] = jnp.where(lane == 3, 1.0, f * cd4)


def _post_body(nf_ref, hacc_ref, cacc_ref, cpad_ref,
               we2_ref, be2_ref, wn1_ref, bn1_ref, wn2_ref, bn2_ref,
               out1_ref, out2_ref):
    d = we2_ref.shape[0]
    hs = hacc_ref[...]
    cacc = cacc_ref[...]
    deg = cacc[:, 3:4]
    nm = jnp.dot(hs, we2_ref[...], preferred_element_type=jnp.float32) + deg * be2_ref[...]
    wn1 = wn1_ref[...]
    pre = (jnp.dot(nf_ref[...], wn1[0:d, :], preferred_element_type=jnp.float32)
           + jnp.dot(nm, wn1[d:2 * d, :], preferred_element_type=jnp.float32)
           + bn1_ref[...])
    h2 = pre * jax.nn.sigmoid(pre)
    out1_ref[...] = jnp.dot(h2, wn2_ref[...], preferred_element_type=jnp.float32) + bn2_ref[...]
    out2_ref[...] = cpad_ref[...] + cacc[:, 0:4]


def _make_gather(n, e, d):
    ew = e // _NW
    nch = ew // _CG
    mesh = plsc.VectorSubcoreMesh(core_axis_name="c", subcore_axis_name="s")
    f32 = jnp.float32

    @functools.partial(
        pl.kernel,
        out_type=[jax.ShapeDtypeStruct((e, d), f32),
                  jax.ShapeDtypeStruct((e, d), f32)]
        + [jax.ShapeDtypeStruct((e,), f32)] * 6,
        mesh=mesh,
        scratch_types=[
            pltpu.VMEM((2, _CG), jnp.int32),
            pltpu.VMEM((2, _CG), jnp.int32),
            pltpu.VMEM((2, _CG, d), f32),
            pltpu.VMEM((2, _CG, d), f32),
            pltpu.VMEM((2, _CG), f32),
            pltpu.VMEM((2, _CG), f32),
            pltpu.VMEM((2, _CG), f32),
            pltpu.VMEM((2, _CG), f32),
            pltpu.VMEM((2, _CG), f32),
            pltpu.VMEM((2, _CG), f32),
            pltpu.SemaphoreType.DMA,
            pltpu.SemaphoreType.DMA,
        ],
    )
    def gather_k(a_hbm, b_hbm, x_hbm, y_hbm, z_hbm, row_hbm, col_hbm,
                 ar_hbm, bc_hbm, xr_hbm, yr_hbm, zr_hbm, xc_hbm, yc_hbm, zc_hbm,
                 idxr_v, idxc_v, abuf, bbuf, xrb, yrb, zrb, xcb, ycb, zcb,
                 semg, semw):
        wid = lax.axis_index("s") * _NC + lax.axis_index("c")
        base = wid * ew
        smalls = [xrb, yrb, zrb, xcb, ycb, zcb]
        souts = [xr_hbm, yr_hbm, zr_hbm, xc_hbm, yc_hbm, zc_hbm]

        def fire(j, p):
            sl = pl.ds(base + j * _CG, _CG)
            pltpu.sync_copy(row_hbm.at[sl], idxr_v.at[p])
            pltpu.sync_copy(col_hbm.at[sl], idxc_v.at[p])
            pltpu.async_copy(a_hbm.at[idxr_v.at[p]], abuf.at[p], semg)
            pltpu.async_copy(b_hbm.at[idxc_v.at[p]], bbuf.at[p], semg)
            for k in range(3):
                t = [x_hbm, y_hbm, z_hbm][k]
                pltpu.async_copy(t.at[idxr_v.at[p]], smalls[k].at[p], semg)
                pltpu.async_copy(t.at[idxc_v.at[p]], smalls[3 + k].at[p], semg)

        def wait_gathers(p):
            pltpu.make_async_copy(a_hbm.at[idxr_v.at[p]], abuf.at[p], semg).wait()
            pltpu.make_async_copy(b_hbm.at[idxc_v.at[p]], bbuf.at[p], semg).wait()
            for k in range(6):
                pltpu.make_async_copy(
                    x_hbm.at[idxr_v.at[p]], smalls[k].at[p], semg).wait()

        def fire_wb(j, p):
            sl = pl.ds(base + j * _CG, _CG)
            pltpu.async_copy(abuf.at[p], ar_hbm.at[sl], semw)
            pltpu.async_copy(bbuf.at[p], bc_hbm.at[sl], semw)
            for k in range(6):
                pltpu.async_copy(smalls[k].at[p], souts[k].at[sl], semw)

        def wait_wb(p):
            sl = pl.ds(base, _CG)
            pltpu.make_async_copy(abuf.at[p], ar_hbm.at[sl], semw).wait()
            pltpu.make_async_copy(bbuf.at[p], bc_hbm.at[sl], semw).wait()
            for k in range(6):
                pltpu.make_async_copy(smalls[k].at[p], souts[k].at[sl], semw).wait()

        fire(0, 0)
        wait_gathers(0)
        fire_wb(0, 0)

        def pair(j2, carry):
            for p, coff in ((1, 1), (0, 2)):
                c = 2 * j2 + coff
                fire(c, p)
                wait_wb(1 - p)
                wait_gathers(p)
                fire_wb(c, p)
            return carry

        lax.fori_loop(0, (nch - 1) // 2, pair, 0)
        wait_wb(0)

    return gather_k


def _make_scatter(n, e, d):
    et = e // _NS
    nch = et // _CG
    nblk = (n + _CG - 1) // _CG
    mesh = plsc.VectorSubcoreMesh(core_axis_name="c", subcore_axis_name="s")
    f32 = jnp.float32

    @functools.partial(
        pl.kernel,
        out_type=[jax.ShapeDtypeStruct((n, d), f32),
                  jax.ShapeDtypeStruct((n, d), f32)],
        mesh=mesh,
        scratch_types=[
            pltpu.VMEM((2, _CG), jnp.int32),
            pltpu.VMEM((2, _CG, d), f32),
            pltpu.VMEM((_CG, d), f32),
            pltpu.VMEM_SHARED((n, d), f32),
            pltpu.SemaphoreType.DMA,
        ],
    )
    def scatter_k(mh_hbm, cm4_hbm, row_hbm, zh_hbm,
                  hout_hbm, cout_hbm,
                  idx_v, mhbuf, hbb, acc, seml):
        cid = lax.axis_index("c")
        sid = lax.axis_index("s")

        def zinit(j, carry):
            b = j * _NS + sid

            @pl.when(b < nblk)
            def _():
                rs = pl.ds(b * _CG, _CG)
                pltpu.sync_copy(zh_hbm.at[rs], hbb)
                pltpu.sync_copy(hbb, acc.at[rs])

            return carry

        lax.fori_loop(0, (nblk + _NS - 1) // _NS, zinit, 0)
        plsc.subcore_barrier()

        ebase = sid * et

        def mk_sweep(src_hbm):
            def fire(j, p):
                sl = pl.ds(ebase + j * _CG, _CG)
                pltpu.async_copy(row_hbm.at[sl], idx_v.at[p], seml)
                pltpu.async_copy(src_hbm.at[sl], mhbuf.at[p], seml)

            def wait_loads(p):
                sl = pl.ds(ebase, _CG)
                pltpu.make_async_copy(row_hbm.at[sl], idx_v.at[p], seml).wait()
                pltpu.make_async_copy(src_hbm.at[sl], mhbuf.at[p], seml).wait()

            def sweep():
                fire(0, 0)

                def pair(j2, carry):
                    for p, coff in ((0, 0), (1, 1)):
                        c = 2 * j2 + coff
                        wait_loads(p)

                        @pl.when(c + 1 < nch)
                        def _():
                            fire(c + 1, 1 - p)

                        pltpu.sync_copy(mhbuf.at[p], acc.at[idx_v.at[p]],
                                        add=True)
                    return carry

                lax.fori_loop(0, (nch + 1) // 2, pair, 0)
            return sweep

        @pl.when(cid == 0)
        def _():
            mk_sweep(mh_hbm)()

        @pl.when(cid == 1)
        def _():
            mk_sweep(cm4_hbm)()

        plsc.subcore_barrier()

        def mk_drain(dst_hbm):
            def drain(j, carry):
                b = j * _NS + sid

                @pl.when(b < nblk)
                def _():
                    rs = pl.ds(b * _CG, _CG)
                    pltpu.sync_copy(acc.at[rs], hbb)
                    pltpu.sync_copy(hbb, dst_hbm.at[rs])

                return carry
            return drain

        @pl.when(cid == 0)
        def _():
            lax.fori_loop(0, (nblk + _NS - 1) // _NS, mk_drain(hout_hbm), 0)

        @pl.when(cid == 1)
        def _():
            lax.fori_loop(0, (nblk + _NS - 1) // _NS, mk_drain(cout_hbm), 0)

    return scatter_k


def kernel(node_features, edge_index, edge_attr, coords,
           We1, be1, We2, be2, Wn1, bn1, Wn2, bn2, Wc1, bc1, Wc2, bc2):
    n, d = node_features.shape
    e = edge_index.shape[1]
    ed = edge_attr.shape[1]

    row = edge_index[0]
    col = edge_index[1]
    ct = coords.T
    cx, cy, cz = ct[0], ct[1], ct[2]
    cpad = jnp.pad(coords, ((0, 0), (0, 1)))
    be1r = be1.reshape(1, d)
    be2r = be2.reshape(1, d)
    bn1r = bn1.reshape(1, d)
    bn2r = bn2.reshape(1, d)
    bc1r = bc1.reshape(1, d)
    bc2r = bc2.reshape(1, 1)

    f32 = jnp.float32
    a, b, w2c, bc1e = pl.pallas_call(
        _prep_body,
        out_shape=[jax.ShapeDtypeStruct((n, d), f32),
                   jax.ShapeDtypeStruct((n, d), f32),
                   jax.ShapeDtypeStruct((d, d), f32),
                   jax.ShapeDtypeStruct((1, d), f32)],
    )(node_features, We1, We2, Wc1, bc1r, be2r)

    ar, bc, xr, yr, zr, xc, yc, zc = _make_gather(n, e, d)(
        a, b, cx, cy, cz, row, col)

    be = 512
    grid_e = e // be
    lane3 = (grid_e, 1, be)
    r3 = lambda v: v.reshape(lane3)
    full = lambda shape: pl.BlockSpec(shape, lambda i: (0,) * len(shape))
    lane_spec = pl.BlockSpec((1, 1, be), lambda i: (i, 0, 0))
    mh, cm4 = pl.pallas_call(
        _edge_body,
        grid=(grid_e,),
        in_specs=[
            pl.BlockSpec((be, d), lambda i: (i, 0)),
            pl.BlockSpec((be, d), lambda i: (i, 0)),
            pl.BlockSpec((be, ed), lambda i: (i, 0)),
            lane_spec, lane_spec, lane_spec, lane_spec, lane_spec, lane_spec,
            full((2 * d + ed, d)),
            full((1, d)),
            full((d, d)),
            full((1, d)),
            full((d, 1)),
            full((1, 1)),
        ],
        out_specs=[
            pl.BlockSpec((be, d), lambda i: (i, 0)),
            pl.BlockSpec((be, d), lambda i: (i, 0)),
        ],
        out_shape=[jax.ShapeDtypeStruct((e, d), f32),
                   jax.ShapeDtypeStruct((e, d), f32)],
    )(ar, bc, edge_attr, r3(xr), r3(yr), r3(zr), r3(xc), r3(yc), r3(zc),
      We1, be1r, w2c, bc1e, Wc2, bc2r)

    zh = jnp.zeros((n, d), f32)
    hacc, cacc = _make_scatter(n, e, d)(mh, cm4, row, zh)

    bn = 1000
    grid_n = n // bn
    nf_new, cnew = pl.pallas_call(
        _post_body,
        grid=(grid_n,),
        in_specs=[
            pl.BlockSpec((bn, d), lambda i: (i, 0)),
            pl.BlockSpec((bn, d), lambda i: (i, 0)),
            pl.BlockSpec((bn, d), lambda i: (i, 0)),
            pl.BlockSpec((bn, 4), lambda i: (i, 0)),
            full((d, d)),
            full((1, d)),
            full((2 * d, d)),
            full((1, d)),
            full((d, d)),
            full((1, d)),
        ],
        out_specs=[
            pl.BlockSpec((bn, d), lambda i: (i, 0)),
            pl.BlockSpec((bn, 4), lambda i: (i, 0)),
        ],
        out_shape=[jax.ShapeDtypeStruct((n, d), f32),
                   jax.ShapeDtypeStruct((n, 4), f32)],
    )(node_features, hacc, cacc, cpad, We2, be2r, Wn1, bn1r, Wn2, bn2r)

    return nf_new, cnew[:, :coords.shape[1]]


# speedup vs baseline: 3.9633x; 1.1759x over previous
import functools

import jax
import jax.numpy as jnp
from jax import lax
from jax.experimental import pallas as pl
from jax.experimental.pallas import tpu as pltpu
from jax.experimental.pallas import tpu_sc as plsc

_NC = 2
_NS = 16
_NW = _NC * _NS
_CG = 80


def _prep_body(nf_ref, we1_ref, we2_ref, wc1_ref, bc1_ref, be2_ref,
               a_ref, b_ref, w2c_ref, bc1e_ref):
    d = we2_ref.shape[0]
    nf = nf_ref[...]
    we1 = we1_ref[...]
    wc1 = wc1_ref[...]
    a_ref[...] = jnp.dot(nf, we1[0:d, :], preferred_element_type=jnp.float32)
    b_ref[...] = jnp.dot(nf, we1[d:2 * d, :], preferred_element_type=jnp.float32)
    w2c_ref[...] = jnp.dot(we2_ref[...], wc1, preferred_element_type=jnp.float32)
    bc1e_ref[...] = bc1_ref[...] + jnp.dot(
        be2_ref[...], wc1, preferred_element_type=jnp.float32)


def _edge_body(ar_ref, bc_ref, ea_ref, xr_ref, yr_ref, zr_ref,
               xc_ref, yc_ref, zc_ref,
               we1_ref, be1_ref, w2c_ref, bc1e_ref, wc2_ref, bc2_ref,
               mh_ref, cm4_ref):
    d = w2c_ref.shape[0]
    ed = ea_ref.shape[1]
    w1e = we1_ref[2 * d:2 * d + ed, :]
    pre = (ar_ref[...] + bc_ref[...]
           + jnp.dot(ea_ref[...], w1e, preferred_element_type=jnp.float32)
           + be1_ref[...])
    h = pre * jax.nn.sigmoid(pre)
    chp = jnp.dot(h.astype(jnp.bfloat16), w2c_ref[...].astype(jnp.bfloat16),
                  preferred_element_type=jnp.float32) + bc1e_ref[...]
    ch = chp * jax.nn.sigmoid(chp)
    mh_ref[...] = h
    s = jnp.dot(ch, wc2_ref[...], preferred_element_type=jnp.float32) + bc2_ref[...]
    cdx = xr_ref[0] - xc_ref[0]
    cdy = yr_ref[0] - yc_ref[0]
    cdz = zr_ref[0] - zc_ref[0]
    cdstack = jnp.concatenate([cdx, cdy, cdz, jnp.zeros_like(cdx)], axis=0)
    rr = lax.broadcasted_iota(jnp.int32, (4, d), 0)
    cc = lax.broadcasted_iota(jnp.int32, (4, d), 1)
    proj = (rr == cc).astype(jnp.float32)
    cd4 = lax.dot_general(cdstack, proj,
                          dimension_numbers=(((0,), (0,)), ((), ())),
                          preferred_element_type=jnp.float32)
    dist = jnp.sqrt(jnp.sum(cd4 * cd4, axis=1, keepdims=True)) + 1e-8
    f = s / dist
    lane = lax.broadcasted_iota(jnp.int32, cd4.shape, 1)
    cm4_ref[...] = jnp.where(lane == 3, 1.0, f * cd4)


def _post_body(nf_ref, hacc0_ref, cacc0_ref, hacc1_ref, cacc1_ref, cpad_ref,
               we2_ref, be2_ref, wn1_ref, bn1_ref, wn2_ref, bn2_ref,
               out1_ref, out2_ref):
    d = we2_ref.shape[0]
    hs = hacc0_ref[...] + hacc1_ref[...]
    cacc = cacc0_ref[...] + cacc1_ref[...]
    deg = cacc[:, 3:4]
    nm = jnp.dot(hs, we2_ref[...], preferred_element_type=jnp.float32) + deg * be2_ref[...]
    wn1 = wn1_ref[...]
    pre = (jnp.dot(nf_ref[...], wn1[0:d, :], preferred_element_type=jnp.float32)
           + jnp.dot(nm, wn1[d:2 * d, :], preferred_element_type=jnp.float32)
           + bn1_ref[...])
    h2 = pre * jax.nn.sigmoid(pre)
    out1_ref[...] = jnp.dot(h2, wn2_ref[...], preferred_element_type=jnp.float32) + bn2_ref[...]
    out2_ref[...] = cpad_ref[...] + cacc[:, 0:4]


def _make_gather(n, e, d, cg=_CG):
    ew = e // _NW
    nch = ew // cg
    mesh = plsc.VectorSubcoreMesh(core_axis_name="c", subcore_axis_name="s")
    f32 = jnp.float32

    @functools.partial(
        pl.kernel,
        out_type=[jax.ShapeDtypeStruct((e, d), f32),
                  jax.ShapeDtypeStruct((e, d), f32)]
        + [jax.ShapeDtypeStruct((e,), f32)] * 6,
        mesh=mesh,
        scratch_types=[
            pltpu.VMEM((2, cg), jnp.int32),
            pltpu.VMEM((2, cg), jnp.int32),
            pltpu.VMEM((2, cg, d), f32),
            pltpu.VMEM((2, cg, d), f32),
            pltpu.VMEM((2, cg), f32),
            pltpu.VMEM((2, cg), f32),
            pltpu.VMEM((2, cg), f32),
            pltpu.VMEM((2, cg), f32),
            pltpu.VMEM((2, cg), f32),
            pltpu.VMEM((2, cg), f32),
            pltpu.SemaphoreType.DMA,
            pltpu.SemaphoreType.DMA,
        ],
    )
    def gather_k(a_hbm, b_hbm, x_hbm, y_hbm, z_hbm, row_hbm, col_hbm,
                 ar_hbm, bc_hbm, xr_hbm, yr_hbm, zr_hbm, xc_hbm, yc_hbm, zc_hbm,
                 idxr_v, idxc_v, abuf, bbuf, xrb, yrb, zrb, xcb, ycb, zcb,
                 semg, semw):
        wid = lax.axis_index("s") * _NC + lax.axis_index("c")
        base = wid * ew
        smalls = [xrb, yrb, zrb, xcb, ycb, zcb]
        souts = [xr_hbm, yr_hbm, zr_hbm, xc_hbm, yc_hbm, zc_hbm]

        def fire(j, p):
            sl = pl.ds(base + j * cg, cg)
            pltpu.sync_copy(row_hbm.at[sl], idxr_v.at[p])
            pltpu.sync_copy(col_hbm.at[sl], idxc_v.at[p])
            pltpu.async_copy(a_hbm.at[idxr_v.at[p]], abuf.at[p], semg)
            pltpu.async_copy(b_hbm.at[idxc_v.at[p]], bbuf.at[p], semg)
            for k in range(3):
                t = [x_hbm, y_hbm, z_hbm][k]
                pltpu.async_copy(t.at[idxr_v.at[p]], smalls[k].at[p], semg)
                pltpu.async_copy(t.at[idxc_v.at[p]], smalls[3 + k].at[p], semg)

        def wait_gathers(p):
            pltpu.make_async_copy(a_hbm.at[idxr_v.at[p]], abuf.at[p], semg).wait()
            pltpu.make_async_copy(b_hbm.at[idxc_v.at[p]], bbuf.at[p], semg).wait()
            for k in range(6):
                pltpu.make_async_copy(
                    x_hbm.at[idxr_v.at[p]], smalls[k].at[p], semg).wait()

        def fire_wb(j, p):
            sl = pl.ds(base + j * cg, cg)
            pltpu.async_copy(abuf.at[p], ar_hbm.at[sl], semw)
            pltpu.async_copy(bbuf.at[p], bc_hbm.at[sl], semw)
            for k in range(6):
                pltpu.async_copy(smalls[k].at[p], souts[k].at[sl], semw)

        def wait_wb(p):
            sl = pl.ds(base, cg)
            pltpu.make_async_copy(abuf.at[p], ar_hbm.at[sl], semw).wait()
            pltpu.make_async_copy(bbuf.at[p], bc_hbm.at[sl], semw).wait()
            for k in range(6):
                pltpu.make_async_copy(smalls[k].at[p], souts[k].at[sl], semw).wait()

        fire(0, 0)
        wait_gathers(0)
        fire_wb(0, 0)

        def pair(j2, carry):
            for p, coff in ((1, 1), (0, 2)):
                c = 2 * j2 + coff
                fire(c, p)
                wait_wb(1 - p)
                wait_gathers(p)
                fire_wb(c, p)
            return carry

        lax.fori_loop(0, (nch - 1) // 2, pair, 0)
        wait_wb(0)

    return gather_k


def _make_scatter(n, e, d):
    et = e // _NS
    nch = et // _CG
    nblk = (n + _CG - 1) // _CG
    mesh = plsc.VectorSubcoreMesh(core_axis_name="c", subcore_axis_name="s")
    f32 = jnp.float32

    @functools.partial(
        pl.kernel,
        out_type=[jax.ShapeDtypeStruct((n, d), f32),
                  jax.ShapeDtypeStruct((n, d), f32)],
        mesh=mesh,
        scratch_types=[
            pltpu.VMEM((2, _CG), jnp.int32),
            pltpu.VMEM((2, _CG, d), f32),
            pltpu.VMEM((_CG, d), f32),
            pltpu.VMEM_SHARED((n, d), f32),
            pltpu.SemaphoreType.DMA,
        ],
    )
    def scatter_k(mh_hbm, cm4_hbm, row_hbm, zh_hbm,
                  hout_hbm, cout_hbm,
                  idx_v, mhbuf, hbb, acc, seml):
        cid = lax.axis_index("c")
        sid = lax.axis_index("s")

        def zinit(j, carry):
            b = j * _NS + sid

            @pl.when(b < nblk)
            def _():
                rs = pl.ds(b * _CG, _CG)
                pltpu.sync_copy(zh_hbm.at[rs], hbb)
                pltpu.sync_copy(hbb, acc.at[rs])

            return carry

        lax.fori_loop(0, (nblk + _NS - 1) // _NS, zinit, 0)
        plsc.subcore_barrier()

        ebase = sid * et

        def mk_sweep(src_hbm):
            def fire(j, p):
                sl = pl.ds(ebase + j * _CG, _CG)
                pltpu.async_copy(row_hbm.at[sl], idx_v.at[p], seml)
                pltpu.async_copy(src_hbm.at[sl], mhbuf.at[p], seml)

            def wait_loads(p):
                sl = pl.ds(ebase, _CG)
                pltpu.make_async_copy(row_hbm.at[sl], idx_v.at[p], seml).wait()
                pltpu.make_async_copy(src_hbm.at[sl], mhbuf.at[p], seml).wait()

            def sweep():
                fire(0, 0)

                def pair(j2, carry):
                    for p, coff in ((0, 0), (1, 1)):
                        c = 2 * j2 + coff

                        def step(c=c, p=p):
                            wait_loads(p)

                            @pl.when(c + 1 < nch)
                            def _():
                                fire(c + 1, 1 - p)

                            pltpu.sync_copy(mhbuf.at[p], acc.at[idx_v.at[p]],
                                            add=True)

                        if coff == 1 and nch % 2 == 1:
                            pl.when(c < nch)(step)
                        else:
                            step()
                    return carry

                lax.fori_loop(0, (nch + 1) // 2, pair, 0)
            return sweep

        @pl.when(cid == 0)
        def _():
            mk_sweep(mh_hbm)()

        @pl.when(cid == 1)
        def _():
            mk_sweep(cm4_hbm)()

        plsc.subcore_barrier()

        def mk_drain(dst_hbm):
            def drain(j, carry):
                b = j * _NS + sid

                @pl.when(b < nblk)
                def _():
                    rs = pl.ds(b * _CG, _CG)
                    pltpu.sync_copy(acc.at[rs], hbb)
                    pltpu.sync_copy(hbb, dst_hbm.at[rs])

                return carry
            return drain

        @pl.when(cid == 0)
        def _():
            lax.fori_loop(0, (nblk + _NS - 1) // _NS, mk_drain(hout_hbm), 0)

        @pl.when(cid == 1)
        def _():
            lax.fori_loop(0, (nblk + _NS - 1) // _NS, mk_drain(cout_hbm), 0)

    return scatter_k


def kernel(node_features, edge_index, edge_attr, coords,
           We1, be1, We2, be2, Wn1, bn1, Wn2, bn2, Wc1, bc1, Wc2, bc2):
    n, d = node_features.shape
    e = edge_index.shape[1]
    ed = edge_attr.shape[1]

    row = edge_index[0]
    col = edge_index[1]
    ct = coords.T
    cx, cy, cz = ct[0], ct[1], ct[2]
    cpad = jnp.pad(coords, ((0, 0), (0, 1)))
    be1r = be1.reshape(1, d)
    be2r = be2.reshape(1, d)
    bn1r = bn1.reshape(1, d)
    bn2r = bn2.reshape(1, d)
    bc1r = bc1.reshape(1, d)
    bc2r = bc2.reshape(1, 1)

    f32 = jnp.float32
    a, b, w2c, bc1e = pl.pallas_call(
        _prep_body,
        out_shape=[jax.ShapeDtypeStruct((n, d), f32),
                   jax.ShapeDtypeStruct((n, d), f32),
                   jax.ShapeDtypeStruct((d, d), f32),
                   jax.ShapeDtypeStruct((1, d), f32)],
    )(node_features, We1, We2, Wc1, bc1r, be2r)

    nsl = 2
    es = e // nsl
    be = 640
    grid_e = es // be
    lane3 = (grid_e, 1, be)
    r3 = lambda v: v.reshape(lane3)
    full = lambda shape: pl.BlockSpec(shape, lambda i: (0,) * len(shape))
    lane_spec = pl.BlockSpec((1, 1, be), lambda i: (i, 0, 0))
    zh = jnp.zeros((n, d), f32)
    gather_f = _make_gather(n, es, d, cg=40)
    scatter_f = _make_scatter(n, es, d)
    parts = []
    for s in range(nsl):
        row_s = lax.slice_in_dim(row, s * es, (s + 1) * es)
        col_s = lax.slice_in_dim(col, s * es, (s + 1) * es)
        ea_s = lax.slice_in_dim(edge_attr, s * es, (s + 1) * es)
        ar, bc, xr, yr, zr, xc, yc, zc = gather_f(a, b, cx, cy, cz,
                                                  row_s, col_s)
        mh, cm4 = pl.pallas_call(
            _edge_body,
            grid=(grid_e,),
            in_specs=[
                pl.BlockSpec((be, d), lambda i: (i, 0)),
                pl.BlockSpec((be, d), lambda i: (i, 0)),
                pl.BlockSpec((be, ed), lambda i: (i, 0)),
                lane_spec, lane_spec, lane_spec,
                lane_spec, lane_spec, lane_spec,
                full((2 * d + ed, d)),
                full((1, d)),
                full((d, d)),
                full((1, d)),
                full((d, 1)),
                full((1, 1)),
            ],
            out_specs=[
                pl.BlockSpec((be, d), lambda i: (i, 0)),
                pl.BlockSpec((be, d), lambda i: (i, 0)),
            ],
            out_shape=[jax.ShapeDtypeStruct((es, d), f32),
                       jax.ShapeDtypeStruct((es, d), f32)],
        )(ar, bc, ea_s, r3(xr), r3(yr), r3(zr), r3(xc), r3(yc), r3(zc),
          We1, be1r, w2c, bc1e, Wc2, bc2r)
        parts.append(scatter_f(mh, cm4, row_s, zh))

    (hacc0, cacc0), (hacc1, cacc1) = parts

    bn = 1000
    grid_n = n // bn
    nf_new, cnew = pl.pallas_call(
        _post_body,
        grid=(grid_n,),
        in_specs=[
            pl.BlockSpec((bn, d), lambda i: (i, 0)),
            pl.BlockSpec((bn, d), lambda i: (i, 0)),
            pl.BlockSpec((bn, d), lambda i: (i, 0)),
            pl.BlockSpec((bn, d), lambda i: (i, 0)),
            pl.BlockSpec((bn, d), lambda i: (i, 0)),
            pl.BlockSpec((bn, 4), lambda i: (i, 0)),
            full((d, d)),
            full((1, d)),
            full((2 * d, d)),
            full((1, d)),
            full((d, d)),
            full((1, d)),
        ],
        out_specs=[
            pl.BlockSpec((bn, d), lambda i: (i, 0)),
            pl.BlockSpec((bn, 4), lambda i: (i, 0)),
        ],
        out_shape=[jax.ShapeDtypeStruct((n, d), f32),
                   jax.ShapeDtypeStruct((n, 4), f32)],
    )(node_features, hacc0, cacc0, hacc1, cacc1, cpad,
      We2, be2r, Wn1, bn1r, Wn2, bn2r)

    return nf_new, cnew[:, :coords.shape[1]]


# speedup vs baseline: 4.2324x; 1.0679x over previous
import functools

import jax
import jax.numpy as jnp
from jax import lax
from jax.experimental import pallas as pl
from jax.experimental.pallas import tpu as pltpu
from jax.experimental.pallas import tpu_sc as plsc

_NC = 2
_NS = 16
_NW = _NC * _NS
_CG = 80


def _prep_body(nf_ref, we1_ref, we2_ref, wc1_ref, bc1_ref, be2_ref,
               a_ref, b_ref, w2c_ref, bc1e_ref):
    d = we2_ref.shape[0]
    nf = nf_ref[...]
    we1 = we1_ref[...]
    wc1 = wc1_ref[...]
    a_ref[...] = jnp.dot(nf, we1[0:d, :], preferred_element_type=jnp.float32)
    b_ref[...] = jnp.dot(nf, we1[d:2 * d, :], preferred_element_type=jnp.float32)
    w2c_ref[...] = jnp.dot(we2_ref[...], wc1, preferred_element_type=jnp.float32)
    bc1e_ref[...] = bc1_ref[...] + jnp.dot(
        be2_ref[...], wc1, preferred_element_type=jnp.float32)


def _edge_body(ar_ref, bc_ref, ea_ref, xr_ref, yr_ref, zr_ref,
               xc_ref, yc_ref, zc_ref,
               we1_ref, be1_ref, w2c_ref, bc1e_ref, wc2_ref, bc2_ref,
               mh_ref, cm4_ref):
    d = w2c_ref.shape[0]
    ed = ea_ref.shape[1]
    w1e = we1_ref[2 * d:2 * d + ed, :]
    pre = (ar_ref[...] + bc_ref[...]
           + jnp.dot(ea_ref[...], w1e, preferred_element_type=jnp.float32)
           + be1_ref[...])
    h = pre * jax.nn.sigmoid(pre)
    chp = jnp.dot(h.astype(jnp.bfloat16), w2c_ref[...].astype(jnp.bfloat16),
                  preferred_element_type=jnp.float32) + bc1e_ref[...]
    ch = chp * jax.nn.sigmoid(chp)
    mh_ref[...] = h
    s = jnp.dot(ch, wc2_ref[...], preferred_element_type=jnp.float32) + bc2_ref[...]
    cdx = xr_ref[0] - xc_ref[0]
    cdy = yr_ref[0] - yc_ref[0]
    cdz = zr_ref[0] - zc_ref[0]
    cdstack = jnp.concatenate([cdx, cdy, cdz, jnp.zeros_like(cdx)], axis=0)
    rr = lax.broadcasted_iota(jnp.int32, (4, d), 0)
    cc = lax.broadcasted_iota(jnp.int32, (4, d), 1)
    proj = (rr == cc).astype(jnp.float32)
    cd4 = lax.dot_general(cdstack, proj,
                          dimension_numbers=(((0,), (0,)), ((), ())),
                          preferred_element_type=jnp.float32)
    dist = jnp.sqrt(jnp.sum(cd4 * cd4, axis=1, keepdims=True)) + 1e-8
    f = s / dist
    lane = lax.broadcasted_iota(jnp.int32, cd4.shape, 1)
    cm4_ref[...] = jnp.where(lane == 3, 1.0, f * cd4)


def _post_body(nf_ref, hacc0_ref, cacc0_ref, hacc1_ref, cacc1_ref, cpad_ref,
               we2_ref, be2_ref, wn1_ref, bn1_ref, wn2_ref, bn2_ref,
               out1_ref, out2_ref):
    d = we2_ref.shape[0]
    hs = hacc0_ref[...] + hacc1_ref[...]
    cacc = cacc0_ref[...] + cacc1_ref[...]
    deg = cacc[:, 3:4]
    nm = jnp.dot(hs, we2_ref[...], preferred_element_type=jnp.float32) + deg * be2_ref[...]
    wn1 = wn1_ref[...]
    pre = (jnp.dot(nf_ref[...], wn1[0:d, :], preferred_element_type=jnp.float32)
           + jnp.dot(nm, wn1[d:2 * d, :], preferred_element_type=jnp.float32)
           + bn1_ref[...])
    h2 = pre * jax.nn.sigmoid(pre)
    out1_ref[...] = jnp.dot(h2, wn2_ref[...], preferred_element_type=jnp.float32) + bn2_ref[...]
    out2_ref[...] = cpad_ref[...] + cacc[:, 0:4]


def _make_gather(n, e, d, cg=_CG):
    ew = e // _NW
    nch = ew // cg
    tail = ew - nch * cg
    assert tail % 8 == 0 and nch % 2 == 1
    mesh = plsc.VectorSubcoreMesh(core_axis_name="c", subcore_axis_name="s")
    f32 = jnp.float32

    @functools.partial(
        pl.kernel,
        out_type=[jax.ShapeDtypeStruct((e, d), f32),
                  jax.ShapeDtypeStruct((e, d), f32)]
        + [jax.ShapeDtypeStruct((e,), f32)] * 6,
        mesh=mesh,
        scratch_types=[
            pltpu.VMEM((2, cg), jnp.int32),
            pltpu.VMEM((2, cg), jnp.int32),
            pltpu.VMEM((2, cg, d), f32),
            pltpu.VMEM((2, cg, d), f32),
            pltpu.VMEM((2, cg), f32),
            pltpu.VMEM((2, cg), f32),
            pltpu.VMEM((2, cg), f32),
            pltpu.VMEM((2, cg), f32),
            pltpu.VMEM((2, cg), f32),
            pltpu.VMEM((2, cg), f32),
            pltpu.SemaphoreType.DMA,
            pltpu.SemaphoreType.DMA,
        ],
    )
    def gather_k(a_hbm, b_hbm, x_hbm, y_hbm, z_hbm, row_hbm, col_hbm,
                 ar_hbm, bc_hbm, xr_hbm, yr_hbm, zr_hbm, xc_hbm, yc_hbm, zc_hbm,
                 idxr_v, idxc_v, abuf, bbuf, xrb, yrb, zrb, xcb, ycb, zcb,
                 semg, semw):
        wid = lax.axis_index("s") * _NC + lax.axis_index("c")
        base = wid * ew
        smalls = [xrb, yrb, zrb, xcb, ycb, zcb]
        souts = [xr_hbm, yr_hbm, zr_hbm, xc_hbm, yc_hbm, zc_hbm]

        def fire(j, p):
            sl = pl.ds(base + j * cg, cg)
            pltpu.sync_copy(row_hbm.at[sl], idxr_v.at[p])
            pltpu.sync_copy(col_hbm.at[sl], idxc_v.at[p])
            pltpu.async_copy(a_hbm.at[idxr_v.at[p]], abuf.at[p], semg)
            pltpu.async_copy(b_hbm.at[idxc_v.at[p]], bbuf.at[p], semg)
            for k in range(3):
                t = [x_hbm, y_hbm, z_hbm][k]
                pltpu.async_copy(t.at[idxr_v.at[p]], smalls[k].at[p], semg)
                pltpu.async_copy(t.at[idxc_v.at[p]], smalls[3 + k].at[p], semg)

        def wait_gathers(p):
            pltpu.make_async_copy(a_hbm.at[idxr_v.at[p]], abuf.at[p], semg).wait()
            pltpu.make_async_copy(b_hbm.at[idxc_v.at[p]], bbuf.at[p], semg).wait()
            for k in range(6):
                pltpu.make_async_copy(
                    x_hbm.at[idxr_v.at[p]], smalls[k].at[p], semg).wait()

        def fire_wb(j, p):
            sl = pl.ds(base + j * cg, cg)
            pltpu.async_copy(abuf.at[p], ar_hbm.at[sl], semw)
            pltpu.async_copy(bbuf.at[p], bc_hbm.at[sl], semw)
            for k in range(6):
                pltpu.async_copy(smalls[k].at[p], souts[k].at[sl], semw)

        def wait_wb(p):
            sl = pl.ds(base, cg)
            pltpu.make_async_copy(abuf.at[p], ar_hbm.at[sl], semw).wait()
            pltpu.make_async_copy(bbuf.at[p], bc_hbm.at[sl], semw).wait()
            for k in range(6):
                pltpu.make_async_copy(smalls[k].at[p], souts[k].at[sl], semw).wait()

        if tail:
            tsl = pl.ds(base + nch * cg, tail)
            tb = pl.ds(0, tail)
            pltpu.sync_copy(row_hbm.at[tsl], idxr_v.at[1, tb])
            pltpu.sync_copy(col_hbm.at[tsl], idxc_v.at[1, tb])
            pltpu.async_copy(a_hbm.at[idxr_v.at[1, tb]], abuf.at[1, tb], semg)
            pltpu.async_copy(b_hbm.at[idxc_v.at[1, tb]], bbuf.at[1, tb], semg)
            for k in range(3):
                t = [x_hbm, y_hbm, z_hbm][k]
                pltpu.async_copy(t.at[idxr_v.at[1, tb]],
                                 smalls[k].at[1, tb], semg)
                pltpu.async_copy(t.at[idxc_v.at[1, tb]],
                                 smalls[3 + k].at[1, tb], semg)
            pltpu.make_async_copy(
                a_hbm.at[idxr_v.at[1, tb]], abuf.at[1, tb], semg).wait()
            pltpu.make_async_copy(
                b_hbm.at[idxc_v.at[1, tb]], bbuf.at[1, tb], semg).wait()
            for k in range(6):
                pltpu.make_async_copy(
                    x_hbm.at[idxr_v.at[1, tb]], smalls[k].at[1, tb], semg).wait()
            pltpu.sync_copy(abuf.at[1, tb], ar_hbm.at[tsl])
            pltpu.sync_copy(bbuf.at[1, tb], bc_hbm.at[tsl])
            for k in range(6):
                pltpu.sync_copy(smalls[k].at[1, tb], souts[k].at[tsl])

        fire(0, 0)
        wait_gathers(0)
        fire_wb(0, 0)

        def pair(j2, carry):
            for p, coff in ((1, 1), (0, 2)):
                c = 2 * j2 + coff
                fire(c, p)
                wait_wb(1 - p)
                wait_gathers(p)
                fire_wb(c, p)
            return carry

        lax.fori_loop(0, (nch - 1) // 2, pair, 0)
        wait_wb(0)

    return gather_k


def _make_scatter(n, e, d):
    et = e // _NS
    nch = et // _CG
    nblk = (n + _CG - 1) // _CG
    mesh = plsc.VectorSubcoreMesh(core_axis_name="c", subcore_axis_name="s")
    f32 = jnp.float32

    @functools.partial(
        pl.kernel,
        out_type=[jax.ShapeDtypeStruct((n, d), f32),
                  jax.ShapeDtypeStruct((n, d), f32)],
        mesh=mesh,
        scratch_types=[
            pltpu.VMEM((2, _CG), jnp.int32),
            pltpu.VMEM((2, _CG, d), f32),
            pltpu.VMEM((_CG, d), f32),
            pltpu.VMEM_SHARED((n, d), f32),
            pltpu.SemaphoreType.DMA,
        ],
    )
    def scatter_k(mh_hbm, cm4_hbm, row_hbm, zh_hbm,
                  hout_hbm, cout_hbm,
                  idx_v, mhbuf, hbb, acc, seml):
        cid = lax.axis_index("c")
        sid = lax.axis_index("s")

        def zinit(j, carry):
            b = j * _NS + sid

            @pl.when(b < nblk)
            def _():
                rs = pl.ds(b * _CG, _CG)
                pltpu.sync_copy(zh_hbm.at[rs], hbb)
                pltpu.sync_copy(hbb, acc.at[rs])

            return carry

        lax.fori_loop(0, (nblk + _NS - 1) // _NS, zinit, 0)
        plsc.subcore_barrier()

        ebase = sid * et

        def mk_sweep(src_hbm):
            def fire(j, p):
                sl = pl.ds(ebase + j * _CG, _CG)
                pltpu.async_copy(row_hbm.at[sl], idx_v.at[p], seml)
                pltpu.async_copy(src_hbm.at[sl], mhbuf.at[p], seml)

            def wait_loads(p):
                sl = pl.ds(ebase, _CG)
                pltpu.make_async_copy(row_hbm.at[sl], idx_v.at[p], seml).wait()
                pltpu.make_async_copy(src_hbm.at[sl], mhbuf.at[p], seml).wait()

            def sweep():
                fire(0, 0)

                def pair(j2, carry):
                    for p, coff in ((0, 0), (1, 1)):
                        c = 2 * j2 + coff

                        def step(c=c, p=p):
                            wait_loads(p)

                            @pl.when(c + 1 < nch)
                            def _():
                                fire(c + 1, 1 - p)

                            pltpu.sync_copy(mhbuf.at[p], acc.at[idx_v.at[p]],
                                            add=True)

                        if coff == 1 and nch % 2 == 1:
                            pl.when(c < nch)(step)
                        else:
                            step()
                    return carry

                lax.fori_loop(0, (nch + 1) // 2, pair, 0)
            return sweep

        @pl.when(cid == 0)
        def _():
            mk_sweep(mh_hbm)()

        @pl.when(cid == 1)
        def _():
            mk_sweep(cm4_hbm)()

        plsc.subcore_barrier()

        def mk_drain(dst_hbm):
            def drain(j, carry):
                b = j * _NS + sid

                @pl.when(b < nblk)
                def _():
                    rs = pl.ds(b * _CG, _CG)
                    pltpu.sync_copy(acc.at[rs], hbb)
                    pltpu.sync_copy(hbb, dst_hbm.at[rs])

                return carry
            return drain

        @pl.when(cid == 0)
        def _():
            lax.fori_loop(0, (nblk + _NS - 1) // _NS, mk_drain(hout_hbm), 0)

        @pl.when(cid == 1)
        def _():
            lax.fori_loop(0, (nblk + _NS - 1) // _NS, mk_drain(cout_hbm), 0)

    return scatter_k


def kernel(node_features, edge_index, edge_attr, coords,
           We1, be1, We2, be2, Wn1, bn1, Wn2, bn2, Wc1, bc1, Wc2, bc2):
    n, d = node_features.shape
    e = edge_index.shape[1]
    ed = edge_attr.shape[1]

    row = edge_index[0]
    col = edge_index[1]
    ct = coords.T
    cx, cy, cz = ct[0], ct[1], ct[2]
    cpad = jnp.pad(coords, ((0, 0), (0, 1)))
    be1r = be1.reshape(1, d)
    be2r = be2.reshape(1, d)
    bn1r = bn1.reshape(1, d)
    bn2r = bn2.reshape(1, d)
    bc1r = bc1.reshape(1, d)
    bc2r = bc2.reshape(1, 1)

    f32 = jnp.float32
    a, b, w2c, bc1e = pl.pallas_call(
        _prep_body,
        out_shape=[jax.ShapeDtypeStruct((n, d), f32),
                   jax.ShapeDtypeStruct((n, d), f32),
                   jax.ShapeDtypeStruct((d, d), f32),
                   jax.ShapeDtypeStruct((1, d), f32)],
    )(node_features, We1, We2, Wc1, bc1r, be2r)

    nsl = 2
    es = e // nsl
    be = 640
    grid_e = es // be
    lane3 = (grid_e, 1, be)
    r3 = lambda v: v.reshape(lane3)
    full = lambda shape: pl.BlockSpec(shape, lambda i: (0,) * len(shape))
    lane_spec = pl.BlockSpec((1, 1, be), lambda i: (i, 0, 0))
    zh = jnp.zeros((n, d), f32)
    gather_f = _make_gather(n, es, d, cg=128)
    scatter_f = _make_scatter(n, es, d)
    parts = []
    for s in range(nsl):
        row_s = lax.slice_in_dim(row, s * es, (s + 1) * es)
        col_s = lax.slice_in_dim(col, s * es, (s + 1) * es)
        ea_s = lax.slice_in_dim(edge_attr, s * es, (s + 1) * es)
        ar, bc, xr, yr, zr, xc, yc, zc = gather_f(a, b, cx, cy, cz,
                                                  row_s, col_s)
        mh, cm4 = pl.pallas_call(
            _edge_body,
            grid=(grid_e,),
            in_specs=[
                pl.BlockSpec((be, d), lambda i: (i, 0)),
                pl.BlockSpec((be, d), lambda i: (i, 0)),
                pl.BlockSpec((be, ed), lambda i: (i, 0)),
                lane_spec, lane_spec, lane_spec,
                lane_spec, lane_spec, lane_spec,
                full((2 * d + ed, d)),
                full((1, d)),
                full((d, d)),
                full((1, d)),
                full((d, 1)),
                full((1, 1)),
            ],
            out_specs=[
                pl.BlockSpec((be, d), lambda i: (i, 0)),
                pl.BlockSpec((be, d), lambda i: (i, 0)),
            ],
            out_shape=[jax.ShapeDtypeStruct((es, d), f32),
                       jax.ShapeDtypeStruct((es, d), f32)],
        )(ar, bc, ea_s, r3(xr), r3(yr), r3(zr), r3(xc), r3(yc), r3(zc),
          We1, be1r, w2c, bc1e, Wc2, bc2r)
        parts.append(scatter_f(mh, cm4, row_s, zh))

    (hacc0, cacc0), (hacc1, cacc1) = parts

    bn = 1000
    grid_n = n // bn
    nf_new, cnew = pl.pallas_call(
        _post_body,
        grid=(grid_n,),
        in_specs=[
            pl.BlockSpec((bn, d), lambda i: (i, 0)),
            pl.BlockSpec((bn, d), lambda i: (i, 0)),
            pl.BlockSpec((bn, d), lambda i: (i, 0)),
            pl.BlockSpec((bn, d), lambda i: (i, 0)),
            pl.BlockSpec((bn, d), lambda i: (i, 0)),
            pl.BlockSpec((bn, 4), lambda i: (i, 0)),
            full((d, d)),
            full((1, d)),
            full((2 * d, d)),
            full((1, d)),
            full((d, d)),
            full((1, d)),
        ],
        out_specs=[
            pl.BlockSpec((bn, d), lambda i: (i, 0)),
            pl.BlockSpec((bn, 4), lambda i: (i, 0)),
        ],
        out_shape=[jax.ShapeDtypeStruct((n, d), f32),
                   jax.ShapeDtypeStruct((n, 4), f32)],
    )(node_features, hacc0, cacc0, hacc1, cacc1, cpad,
      We2, be2r, Wn1, bn1r, Wn2, bn2r)

    return nf_new, cnew[:, :coords.shape[1]]


# speedup vs baseline: 4.3303x; 1.0231x over previous
import functools

import jax
import jax.numpy as jnp
from jax import lax
from jax.experimental import pallas as pl
from jax.experimental.pallas import tpu as pltpu
from jax.experimental.pallas import tpu_sc as plsc

_NC = 2
_NS = 16
_NW = _NC * _NS
_CG = 80


def _prep_body(nf_ref, we1_ref, we2_ref, wc1_ref, bc1_ref, be2_ref,
               a_ref, b_ref, w2c_ref, bc1e_ref):
    d = we2_ref.shape[0]
    nf = nf_ref[...]
    we1 = we1_ref[...]
    wc1 = wc1_ref[...]
    a_ref[...] = jnp.dot(nf, we1[0:d, :], preferred_element_type=jnp.float32)
    b_ref[...] = jnp.dot(nf, we1[d:2 * d, :], preferred_element_type=jnp.float32)
    w2c_ref[...] = jnp.dot(we2_ref[...], wc1, preferred_element_type=jnp.float32)
    bc1e_ref[...] = bc1_ref[...] + jnp.dot(
        be2_ref[...], wc1, preferred_element_type=jnp.float32)


def _edge_body(ar_ref, bc_ref, ea_ref, xr_ref, yr_ref, zr_ref,
               xc_ref, yc_ref, zc_ref,
               we1_ref, be1_ref, w2c_ref, bc1e_ref, wc2_ref, bc2_ref,
               mh_ref, cm4_ref):
    d = w2c_ref.shape[0]
    ed = ea_ref.shape[1]
    w1e = we1_ref[2 * d:2 * d + ed, :]
    pre = (ar_ref[...] + bc_ref[...]
           + jnp.dot(ea_ref[...], w1e, preferred_element_type=jnp.float32)
           + be1_ref[...])
    h = pre * jax.nn.sigmoid(pre)
    chp = jnp.dot(h.astype(jnp.bfloat16), w2c_ref[...].astype(jnp.bfloat16),
                  preferred_element_type=jnp.float32) + bc1e_ref[...]
    ch = chp * jax.nn.sigmoid(chp)
    mh_ref[...] = h
    s = jnp.dot(ch, wc2_ref[...], preferred_element_type=jnp.float32) + bc2_ref[...]
    cdx = xr_ref[0] - xc_ref[0]
    cdy = yr_ref[0] - yc_ref[0]
    cdz = zr_ref[0] - zc_ref[0]
    cdstack = jnp.concatenate([cdx, cdy, cdz, jnp.zeros_like(cdx)], axis=0)
    rr = lax.broadcasted_iota(jnp.int32, (4, d), 0)
    cc = lax.broadcasted_iota(jnp.int32, (4, d), 1)
    proj = (rr == cc).astype(jnp.float32)
    cd4 = lax.dot_general(cdstack, proj,
                          dimension_numbers=(((0,), (0,)), ((), ())),
                          preferred_element_type=jnp.float32)
    dist = jnp.sqrt(jnp.sum(cd4 * cd4, axis=1, keepdims=True)) + 1e-8
    f = s / dist
    lane = lax.broadcasted_iota(jnp.int32, cd4.shape, 1)
    cm4_ref[...] = jnp.where(lane == 3, 1.0, f * cd4)


def _post_body(nf_ref, hacc0_ref, cacc0_ref, hacc1_ref, cacc1_ref, cpad_ref,
               we2_ref, be2_ref, wn1_ref, bn1_ref, wn2_ref, bn2_ref,
               out1_ref, out2_ref):
    d = we2_ref.shape[0]
    hs = hacc0_ref[...] + hacc1_ref[...]
    cacc = cacc0_ref[...] + cacc1_ref[...]
    deg = cacc[:, 3:4]
    nm = jnp.dot(hs, we2_ref[...], preferred_element_type=jnp.float32) + deg * be2_ref[...]
    wn1 = wn1_ref[...]
    pre = (jnp.dot(nf_ref[...], wn1[0:d, :], preferred_element_type=jnp.float32)
           + jnp.dot(nm, wn1[d:2 * d, :], preferred_element_type=jnp.float32)
           + bn1_ref[...])
    h2 = pre * jax.nn.sigmoid(pre)
    out1_ref[...] = jnp.dot(h2, wn2_ref[...], preferred_element_type=jnp.float32) + bn2_ref[...]
    out2_ref[...] = cpad_ref[...] + cacc[:, 0:4]


def _make_gather(n, e, d, cg=_CG):
    ew = e // _NW
    nch = ew // cg
    tail = ew - nch * cg
    assert tail % 8 == 0 and nch % 2 == 1
    mesh = plsc.VectorSubcoreMesh(core_axis_name="c", subcore_axis_name="s")
    f32 = jnp.float32

    @functools.partial(
        pl.kernel,
        out_type=[jax.ShapeDtypeStruct((e, d), f32),
                  jax.ShapeDtypeStruct((e, d), f32)]
        + [jax.ShapeDtypeStruct((e,), f32)] * 6,
        mesh=mesh,
        scratch_types=[
            pltpu.VMEM((2, cg), jnp.int32),
            pltpu.VMEM((2, cg), jnp.int32),
            pltpu.VMEM((2, cg, d), f32),
            pltpu.VMEM((2, cg, d), f32),
            pltpu.VMEM((2, cg), f32),
            pltpu.VMEM((2, cg), f32),
            pltpu.VMEM((2, cg), f32),
            pltpu.VMEM((2, cg), f32),
            pltpu.VMEM((2, cg), f32),
            pltpu.VMEM((2, cg), f32),
            pltpu.SemaphoreType.DMA,
            pltpu.SemaphoreType.DMA,
        ],
    )
    def gather_k(a_hbm, b_hbm, x_hbm, y_hbm, z_hbm, row_hbm, col_hbm,
                 ar_hbm, bc_hbm, xr_hbm, yr_hbm, zr_hbm, xc_hbm, yc_hbm, zc_hbm,
                 idxr_v, idxc_v, abuf, bbuf, xrb, yrb, zrb, xcb, ycb, zcb,
                 semg, semw):
        wid = lax.axis_index("s") * _NC + lax.axis_index("c")
        base = wid * ew
        smalls = [xrb, yrb, zrb, xcb, ycb, zcb]
        souts = [xr_hbm, yr_hbm, zr_hbm, xc_hbm, yc_hbm, zc_hbm]

        def fire(j, p):
            sl = pl.ds(base + j * cg, cg)
            pltpu.sync_copy(row_hbm.at[sl], idxr_v.at[p])
            pltpu.sync_copy(col_hbm.at[sl], idxc_v.at[p])
            pltpu.async_copy(a_hbm.at[idxr_v.at[p]], abuf.at[p], semg)
            pltpu.async_copy(b_hbm.at[idxc_v.at[p]], bbuf.at[p], semg)
            for k in range(3):
                t = [x_hbm, y_hbm, z_hbm][k]
                pltpu.async_copy(t.at[idxr_v.at[p]], smalls[k].at[p], semg)
                pltpu.async_copy(t.at[idxc_v.at[p]], smalls[3 + k].at[p], semg)

        def wait_gathers(p):
            pltpu.make_async_copy(a_hbm.at[idxr_v.at[p]], abuf.at[p], semg).wait()
            pltpu.make_async_copy(b_hbm.at[idxc_v.at[p]], bbuf.at[p], semg).wait()
            for k in range(6):
                pltpu.make_async_copy(
                    x_hbm.at[idxr_v.at[p]], smalls[k].at[p], semg).wait()

        def fire_wb(j, p):
            sl = pl.ds(base + j * cg, cg)
            pltpu.async_copy(abuf.at[p], ar_hbm.at[sl], semw)
            pltpu.async_copy(bbuf.at[p], bc_hbm.at[sl], semw)
            for k in range(6):
                pltpu.async_copy(smalls[k].at[p], souts[k].at[sl], semw)

        def wait_wb(p):
            sl = pl.ds(base, cg)
            pltpu.make_async_copy(abuf.at[p], ar_hbm.at[sl], semw).wait()
            pltpu.make_async_copy(bbuf.at[p], bc_hbm.at[sl], semw).wait()
            for k in range(6):
                pltpu.make_async_copy(smalls[k].at[p], souts[k].at[sl], semw).wait()

        if tail:
            tsl = pl.ds(base + nch * cg, tail)
            tb = pl.ds(0, tail)
            pltpu.sync_copy(row_hbm.at[tsl], idxr_v.at[1, tb])
            pltpu.sync_copy(col_hbm.at[tsl], idxc_v.at[1, tb])
            pltpu.async_copy(a_hbm.at[idxr_v.at[1, tb]], abuf.at[1, tb], semg)
            pltpu.async_copy(b_hbm.at[idxc_v.at[1, tb]], bbuf.at[1, tb], semg)
            for k in range(3):
                t = [x_hbm, y_hbm, z_hbm][k]
                pltpu.async_copy(t.at[idxr_v.at[1, tb]],
                                 smalls[k].at[1, tb], semg)
                pltpu.async_copy(t.at[idxc_v.at[1, tb]],
                                 smalls[3 + k].at[1, tb], semg)
            pltpu.make_async_copy(
                a_hbm.at[idxr_v.at[1, tb]], abuf.at[1, tb], semg).wait()
            pltpu.make_async_copy(
                b_hbm.at[idxc_v.at[1, tb]], bbuf.at[1, tb], semg).wait()
            for k in range(6):
                pltpu.make_async_copy(
                    x_hbm.at[idxr_v.at[1, tb]], smalls[k].at[1, tb], semg).wait()
            pltpu.sync_copy(abuf.at[1, tb], ar_hbm.at[tsl])
            pltpu.sync_copy(bbuf.at[1, tb], bc_hbm.at[tsl])
            for k in range(6):
                pltpu.sync_copy(smalls[k].at[1, tb], souts[k].at[tsl])

        fire(0, 0)
        wait_gathers(0)
        fire_wb(0, 0)

        def pair(j2, carry):
            for p, coff in ((1, 1), (0, 2)):
                c = 2 * j2 + coff
                fire(c, p)
                wait_wb(1 - p)
                wait_gathers(p)
                fire_wb(c, p)
            return carry

        lax.fori_loop(0, (nch - 1) // 2, pair, 0)
        wait_wb(0)

    return gather_k


def _make_scatter(n, e, d, cs=128):
    et = e // _NS
    nch = et // cs
    tail = et - nch * cs
    assert tail % 8 == 0
    nblk = (n + _CG - 1) // _CG
    mesh = plsc.VectorSubcoreMesh(core_axis_name="c", subcore_axis_name="s")
    f32 = jnp.float32

    @functools.partial(
        pl.kernel,
        out_type=[jax.ShapeDtypeStruct((n, d), f32),
                  jax.ShapeDtypeStruct((n, d), f32)],
        mesh=mesh,
        scratch_types=[
            pltpu.VMEM((2, cs), jnp.int32),
            pltpu.VMEM((2, cs, d), f32),
            pltpu.VMEM((max(tail, 8),), jnp.int32),
            pltpu.VMEM((max(tail, 8), d), f32),
            pltpu.VMEM((_CG, d), f32),
            pltpu.VMEM_SHARED((n, d), f32),
            pltpu.SemaphoreType.DMA,
        ],
    )
    def scatter_k(mh_hbm, cm4_hbm, row_hbm, zh_hbm,
                  hout_hbm, cout_hbm,
                  idx_v, mhbuf, idx_t, mh_t, hbb, acc, seml):
        cid = lax.axis_index("c")
        sid = lax.axis_index("s")

        def zinit(j, carry):
            b = j * _NS + sid

            @pl.when(b < nblk)
            def _():
                rs = pl.ds(b * _CG, _CG)
                pltpu.sync_copy(zh_hbm.at[rs], hbb)
                pltpu.sync_copy(hbb, acc.at[rs])

            return carry

        lax.fori_loop(0, (nblk + _NS - 1) // _NS, zinit, 0)
        plsc.subcore_barrier()

        ebase = sid * et

        def mk_sweep(src_hbm):
            def fire(j, p):
                sl = pl.ds(ebase + j * cs, cs)
                pltpu.async_copy(row_hbm.at[sl], idx_v.at[p], seml)
                pltpu.async_copy(src_hbm.at[sl], mhbuf.at[p], seml)

            def wait_loads(p):
                sl = pl.ds(ebase, cs)
                pltpu.make_async_copy(row_hbm.at[sl], idx_v.at[p], seml).wait()
                pltpu.make_async_copy(src_hbm.at[sl], mhbuf.at[p], seml).wait()

            def sweep():
                if tail:
                    tsl = pl.ds(ebase + nch * cs, tail)
                    pltpu.sync_copy(row_hbm.at[tsl], idx_t)
                    pltpu.sync_copy(src_hbm.at[tsl], mh_t)
                    pltpu.sync_copy(mh_t, acc.at[idx_t], add=True)
                fire(0, 0)

                def pair(j2, carry):
                    for p, coff in ((0, 0), (1, 1)):
                        c = 2 * j2 + coff

                        def step(c=c, p=p):
                            wait_loads(p)

                            @pl.when(c + 1 < nch)
                            def _():
                                fire(c + 1, 1 - p)

                            pltpu.sync_copy(mhbuf.at[p], acc.at[idx_v.at[p]],
                                            add=True)

                        if coff == 1 and nch % 2 == 1:
                            pl.when(c < nch)(step)
                        else:
                            step()
                    return carry

                lax.fori_loop(0, (nch + 1) // 2, pair, 0)
            return sweep

        @pl.when(cid == 0)
        def _():
            mk_sweep(mh_hbm)()

        @pl.when(cid == 1)
        def _():
            mk_sweep(cm4_hbm)()

        plsc.subcore_barrier()

        def mk_drain(dst_hbm):
            def drain(j, carry):
                b = j * _NS + sid

                @pl.when(b < nblk)
                def _():
                    rs = pl.ds(b * _CG, _CG)
                    pltpu.sync_copy(acc.at[rs], hbb)
                    pltpu.sync_copy(hbb, dst_hbm.at[rs])

                return carry
            return drain

        @pl.when(cid == 0)
        def _():
            lax.fori_loop(0, (nblk + _NS - 1) // _NS, mk_drain(hout_hbm), 0)

        @pl.when(cid == 1)
        def _():
            lax.fori_loop(0, (nblk + _NS - 1) // _NS, mk_drain(cout_hbm), 0)

    return scatter_k


def kernel(node_features, edge_index, edge_attr, coords,
           We1, be1, We2, be2, Wn1, bn1, Wn2, bn2, Wc1, bc1, Wc2, bc2):
    n, d = node_features.shape
    e = edge_index.shape[1]
    ed = edge_attr.shape[1]

    row = edge_index[0]
    col = edge_index[1]
    ct = coords.T
    cx, cy, cz = ct[0], ct[1], ct[2]
    cpad = jnp.pad(coords, ((0, 0), (0, 1)))
    be1r = be1.reshape(1, d)
    be2r = be2.reshape(1, d)
    bn1r = bn1.reshape(1, d)
    bn2r = bn2.reshape(1, d)
    bc1r = bc1.reshape(1, d)
    bc2r = bc2.reshape(1, 1)

    f32 = jnp.float32
    a, b, w2c, bc1e = pl.pallas_call(
        _prep_body,
        out_shape=[jax.ShapeDtypeStruct((n, d), f32),
                   jax.ShapeDtypeStruct((n, d), f32),
                   jax.ShapeDtypeStruct((d, d), f32),
                   jax.ShapeDtypeStruct((1, d), f32)],
    )(node_features, We1, We2, Wc1, bc1r, be2r)

    nsl = 2
    es = e // nsl
    be = 640
    grid_e = es // be
    lane3 = (grid_e, 1, be)
    r3 = lambda v: v.reshape(lane3)
    full = lambda shape: pl.BlockSpec(shape, lambda i: (0,) * len(shape))
    lane_spec = pl.BlockSpec((1, 1, be), lambda i: (i, 0, 0))
    zh = jnp.zeros((n, d), f32)
    gather_f = _make_gather(n, es, d, cg=128)
    scatter_f = _make_scatter(n, es, d)
    parts = []
    for s in range(nsl):
        row_s = lax.slice_in_dim(row, s * es, (s + 1) * es)
        col_s = lax.slice_in_dim(col, s * es, (s + 1) * es)
        ea_s = lax.slice_in_dim(edge_attr, s * es, (s + 1) * es)
        ar, bc, xr, yr, zr, xc, yc, zc = gather_f(a, b, cx, cy, cz,
                                                  row_s, col_s)
        mh, cm4 = pl.pallas_call(
            _edge_body,
            grid=(grid_e,),
            in_specs=[
                pl.BlockSpec((be, d), lambda i: (i, 0)),
                pl.BlockSpec((be, d), lambda i: (i, 0)),
                pl.BlockSpec((be, ed), lambda i: (i, 0)),
                lane_spec, lane_spec, lane_spec,
                lane_spec, lane_spec, lane_spec,
                full((2 * d + ed, d)),
                full((1, d)),
                full((d, d)),
                full((1, d)),
                full((d, 1)),
                full((1, 1)),
            ],
            out_specs=[
                pl.BlockSpec((be, d), lambda i: (i, 0)),
                pl.BlockSpec((be, d), lambda i: (i, 0)),
            ],
            out_shape=[jax.ShapeDtypeStruct((es, d), f32),
                       jax.ShapeDtypeStruct((es, d), f32)],
        )(ar, bc, ea_s, r3(xr), r3(yr), r3(zr), r3(xc), r3(yc), r3(zc),
          We1, be1r, w2c, bc1e, Wc2, bc2r)
        parts.append(scatter_f(mh, cm4, row_s, zh))

    (hacc0, cacc0), (hacc1, cacc1) = parts

    bn = 1000
    grid_n = n // bn
    nf_new, cnew = pl.pallas_call(
        _post_body,
        grid=(grid_n,),
        in_specs=[
            pl.BlockSpec((bn, d), lambda i: (i, 0)),
            pl.BlockSpec((bn, d), lambda i: (i, 0)),
            pl.BlockSpec((bn, d), lambda i: (i, 0)),
            pl.BlockSpec((bn, d), lambda i: (i, 0)),
            pl.BlockSpec((bn, d), lambda i: (i, 0)),
            pl.BlockSpec((bn, 4), lambda i: (i, 0)),
            full((d, d)),
            full((1, d)),
            full((2 * d, d)),
            full((1, d)),
            full((d, d)),
            full((1, d)),
        ],
        out_specs=[
            pl.BlockSpec((bn, d), lambda i: (i, 0)),
            pl.BlockSpec((bn, 4), lambda i: (i, 0)),
        ],
        out_shape=[jax.ShapeDtypeStruct((n, d), f32),
                   jax.ShapeDtypeStruct((n, 4), f32)],
    )(node_features, hacc0, cacc0, hacc1, cacc1, cpad,
      We2, be2r, Wn1, bn1r, Wn2, bn2r)

    return nf_new, cnew[:, :coords.shape[1]]


# speedup vs baseline: 4.7142x; 1.0886x over previous
import functools

import jax
import jax.numpy as jnp
from jax import lax
from jax.experimental import pallas as pl
from jax.experimental.pallas import tpu as pltpu
from jax.experimental.pallas import tpu_sc as plsc

_NC = 2
_NS = 16
_NW = _NC * _NS
_CG = 80


def _prep_body(nf_ref, we1_ref, we2_ref, wc1_ref, bc1_ref, be2_ref,
               a_ref, b_ref, w2c_ref, bc1e_ref):
    d = we2_ref.shape[0]
    nf = nf_ref[...]
    we1 = we1_ref[...]
    wc1 = wc1_ref[...]
    a_ref[...] = jnp.dot(nf, we1[0:d, :], preferred_element_type=jnp.float32)
    b_ref[...] = jnp.dot(nf, we1[d:2 * d, :], preferred_element_type=jnp.float32)
    w2c_ref[...] = jnp.dot(we2_ref[...], wc1, preferred_element_type=jnp.float32)
    bc1e_ref[...] = bc1_ref[...] + jnp.dot(
        be2_ref[...], wc1, preferred_element_type=jnp.float32)


def _edge_body(ar_ref, bc_ref, ea_ref, xr_ref, yr_ref, zr_ref,
               xc_ref, yc_ref, zc_ref,
               we1_ref, be1_ref, w2c_ref, bc1e_ref, wc2_ref, bc2_ref,
               mh_ref, cm4_ref):
    d = w2c_ref.shape[0]
    ed = ea_ref.shape[1]
    w1e = we1_ref[2 * d:2 * d + ed, :]
    pre = (ar_ref[...] + bc_ref[...]
           + jnp.dot(ea_ref[...], w1e, preferred_element_type=jnp.float32)
           + be1_ref[...])
    h = pre * jax.nn.sigmoid(pre)
    chp = jnp.dot(h.astype(jnp.bfloat16), w2c_ref[...].astype(jnp.bfloat16),
                  preferred_element_type=jnp.float32) + bc1e_ref[...]
    ch = chp * jax.nn.sigmoid(chp)
    mh_ref[...] = h
    s = jnp.dot(ch, wc2_ref[...], preferred_element_type=jnp.float32) + bc2_ref[...]
    cdx = xr_ref[0] - xc_ref[0]
    cdy = yr_ref[0] - yc_ref[0]
    cdz = zr_ref[0] - zc_ref[0]
    cdstack = jnp.concatenate([cdx, cdy, cdz, jnp.zeros_like(cdx)], axis=0)
    rr = lax.broadcasted_iota(jnp.int32, (4, d), 0)
    cc = lax.broadcasted_iota(jnp.int32, (4, d), 1)
    proj = (rr == cc).astype(jnp.float32)
    cd4 = lax.dot_general(cdstack, proj,
                          dimension_numbers=(((0,), (0,)), ((), ())),
                          preferred_element_type=jnp.float32)
    dist = jnp.sqrt(jnp.sum(cd4 * cd4, axis=1, keepdims=True)) + 1e-8
    f = s / dist
    lane = lax.broadcasted_iota(jnp.int32, cd4.shape, 1)
    cm4_ref[...] = jnp.where(lane == 3, 1.0, f * cd4)


def _post_body(nf_ref, hacc0_ref, cacc0_ref, hacc1_ref, cacc1_ref, cpad_ref,
               we2_ref, be2_ref, wn1_ref, bn1_ref, wn2_ref, bn2_ref,
               out1_ref, out2_ref):
    d = we2_ref.shape[0]
    hs = hacc0_ref[...] + hacc1_ref[...]
    cacc = cacc0_ref[...] + cacc1_ref[...]
    deg = cacc[:, 3:4]
    nm = jnp.dot(hs, we2_ref[...], preferred_element_type=jnp.float32) + deg * be2_ref[...]
    wn1 = wn1_ref[...]
    pre = (jnp.dot(nf_ref[...], wn1[0:d, :], preferred_element_type=jnp.float32)
           + jnp.dot(nm, wn1[d:2 * d, :], preferred_element_type=jnp.float32)
           + bn1_ref[...])
    h2 = pre * jax.nn.sigmoid(pre)
    out1_ref[...] = jnp.dot(h2, wn2_ref[...], preferred_element_type=jnp.float32) + bn2_ref[...]
    out2_ref[...] = cpad_ref[...] + cacc[:, 0:4]


def _make_gather(n, e, d, cg=_CG):
    ew = e // _NW
    nch = ew // cg
    tail = ew - nch * cg
    assert tail % 8 == 0 and nch % 2 == 1
    mesh = plsc.VectorSubcoreMesh(core_axis_name="c", subcore_axis_name="s")
    f32 = jnp.float32

    @functools.partial(
        pl.kernel,
        out_type=[jax.ShapeDtypeStruct((e, d), f32),
                  jax.ShapeDtypeStruct((e, d), f32)]
        + [jax.ShapeDtypeStruct((e,), f32)] * 6,
        mesh=mesh,
        scratch_types=[
            pltpu.VMEM((2, cg), jnp.int32),
            pltpu.VMEM((2, cg), jnp.int32),
            pltpu.VMEM((2, cg, d), f32),
            pltpu.VMEM((2, cg, d), f32),
            pltpu.VMEM((2, cg), f32),
            pltpu.VMEM((2, cg), f32),
            pltpu.VMEM((2, cg), f32),
            pltpu.VMEM((2, cg), f32),
            pltpu.VMEM((2, cg), f32),
            pltpu.VMEM((2, cg), f32),
            pltpu.SemaphoreType.DMA,
            pltpu.SemaphoreType.DMA,
        ],
    )
    def gather_k(a_hbm, b_hbm, x_hbm, y_hbm, z_hbm, row_hbm, col_hbm,
                 ar_hbm, bc_hbm, xr_hbm, yr_hbm, zr_hbm, xc_hbm, yc_hbm, zc_hbm,
                 idxr_v, idxc_v, abuf, bbuf, xrb, yrb, zrb, xcb, ycb, zcb,
                 semg, semw):
        wid = lax.axis_index("s") * _NC + lax.axis_index("c")
        base = wid * ew
        smalls = [xrb, yrb, zrb, xcb, ycb, zcb]
        souts = [xr_hbm, yr_hbm, zr_hbm, xc_hbm, yc_hbm, zc_hbm]

        def fire(j, p):
            sl = pl.ds(base + j * cg, cg)
            pltpu.sync_copy(row_hbm.at[sl], idxr_v.at[p])
            pltpu.sync_copy(col_hbm.at[sl], idxc_v.at[p])
            pltpu.async_copy(a_hbm.at[idxr_v.at[p]], abuf.at[p], semg)
            pltpu.async_copy(b_hbm.at[idxc_v.at[p]], bbuf.at[p], semg)
            for k in range(3):
                t = [x_hbm, y_hbm, z_hbm][k]
                pltpu.async_copy(t.at[idxr_v.at[p]], smalls[k].at[p], semg)
                pltpu.async_copy(t.at[idxc_v.at[p]], smalls[3 + k].at[p], semg)

        def wait_gathers(p):
            pltpu.make_async_copy(a_hbm.at[idxr_v.at[p]], abuf.at[p], semg).wait()
            pltpu.make_async_copy(b_hbm.at[idxc_v.at[p]], bbuf.at[p], semg).wait()
            for k in range(6):
                pltpu.make_async_copy(
                    x_hbm.at[idxr_v.at[p]], smalls[k].at[p], semg).wait()

        def fire_wb(j, p):
            sl = pl.ds(base + j * cg, cg)
            pltpu.async_copy(abuf.at[p], ar_hbm.at[sl], semw)
            pltpu.async_copy(bbuf.at[p], bc_hbm.at[sl], semw)
            for k in range(6):
                pltpu.async_copy(smalls[k].at[p], souts[k].at[sl], semw)

        def wait_wb(p):
            sl = pl.ds(base, cg)
            pltpu.make_async_copy(abuf.at[p], ar_hbm.at[sl], semw).wait()
            pltpu.make_async_copy(bbuf.at[p], bc_hbm.at[sl], semw).wait()
            for k in range(6):
                pltpu.make_async_copy(smalls[k].at[p], souts[k].at[sl], semw).wait()

        if tail:
            tsl = pl.ds(base + nch * cg, tail)
            tb = pl.ds(0, tail)
            pltpu.sync_copy(row_hbm.at[tsl], idxr_v.at[1, tb])
            pltpu.sync_copy(col_hbm.at[tsl], idxc_v.at[1, tb])
            pltpu.async_copy(a_hbm.at[idxr_v.at[1, tb]], abuf.at[1, tb], semg)
            pltpu.async_copy(b_hbm.at[idxc_v.at[1, tb]], bbuf.at[1, tb], semg)
            for k in range(3):
                t = [x_hbm, y_hbm, z_hbm][k]
                pltpu.async_copy(t.at[idxr_v.at[1, tb]],
                                 smalls[k].at[1, tb], semg)
                pltpu.async_copy(t.at[idxc_v.at[1, tb]],
                                 smalls[3 + k].at[1, tb], semg)
            pltpu.make_async_copy(
                a_hbm.at[idxr_v.at[1, tb]], abuf.at[1, tb], semg).wait()
            pltpu.make_async_copy(
                b_hbm.at[idxc_v.at[1, tb]], bbuf.at[1, tb], semg).wait()
            for k in range(6):
                pltpu.make_async_copy(
                    x_hbm.at[idxr_v.at[1, tb]], smalls[k].at[1, tb], semg).wait()
            pltpu.sync_copy(abuf.at[1, tb], ar_hbm.at[tsl])
            pltpu.sync_copy(bbuf.at[1, tb], bc_hbm.at[tsl])
            for k in range(6):
                pltpu.sync_copy(smalls[k].at[1, tb], souts[k].at[tsl])

        fire(0, 0)
        wait_gathers(0)
        fire_wb(0, 0)

        def pair(j2, carry):
            for p, coff in ((1, 1), (0, 2)):
                c = 2 * j2 + coff
                fire(c, p)
                wait_wb(1 - p)
                wait_gathers(p)
                fire_wb(c, p)
            return carry

        lax.fori_loop(0, (nch - 1) // 2, pair, 0)
        wait_wb(0)

    return gather_k


def _make_scatter(n, e, d, cs=128):
    et = e // _NS
    nch = et // cs
    tail = et - nch * cs
    assert tail % 8 == 0
    nblk = (n + _CG - 1) // _CG
    mesh = plsc.VectorSubcoreMesh(core_axis_name="c", subcore_axis_name="s")
    f32 = jnp.float32

    @functools.partial(
        pl.kernel,
        out_type=[jax.ShapeDtypeStruct((n, d), f32),
                  jax.ShapeDtypeStruct((n, d), f32)],
        mesh=mesh,
        scratch_types=[
            pltpu.VMEM((2, cs), jnp.int32),
            pltpu.VMEM((2, cs, d), f32),
            pltpu.VMEM((max(tail, 8),), jnp.int32),
            pltpu.VMEM((max(tail, 8), d), f32),
            pltpu.VMEM((_CG, d), f32),
            pltpu.VMEM_SHARED((n, d), f32),
            pltpu.SemaphoreType.DMA,
        ],
    )
    def scatter_k(mh_hbm, cm4_hbm, row_hbm, zh_hbm,
                  hout_hbm, cout_hbm,
                  idx_v, mhbuf, idx_t, mh_t, hbb, acc, seml):
        cid = lax.axis_index("c")
        sid = lax.axis_index("s")

        def zinit(j, carry):
            b = j * _NS + sid

            @pl.when(b < nblk)
            def _():
                rs = pl.ds(b * _CG, _CG)
                pltpu.sync_copy(zh_hbm.at[rs], hbb)
                pltpu.sync_copy(hbb, acc.at[rs])

            return carry

        lax.fori_loop(0, (nblk + _NS - 1) // _NS, zinit, 0)
        plsc.subcore_barrier()

        ebase = sid * et

        def mk_sweep(src_hbm):
            def fire(j, p):
                sl = pl.ds(ebase + j * cs, cs)
                pltpu.async_copy(row_hbm.at[sl], idx_v.at[p], seml)
                pltpu.async_copy(src_hbm.at[sl], mhbuf.at[p], seml)

            def wait_loads(p):
                sl = pl.ds(ebase, cs)
                pltpu.make_async_copy(row_hbm.at[sl], idx_v.at[p], seml).wait()
                pltpu.make_async_copy(src_hbm.at[sl], mhbuf.at[p], seml).wait()

            def sweep():
                if tail:
                    tsl = pl.ds(ebase + nch * cs, tail)
                    pltpu.sync_copy(row_hbm.at[tsl], idx_t)
                    pltpu.sync_copy(src_hbm.at[tsl], mh_t)
                    pltpu.sync_copy(mh_t, acc.at[idx_t], add=True)
                fire(0, 0)

                def pair(j2, carry):
                    for p, coff in ((0, 0), (1, 1)):
                        c = 2 * j2 + coff

                        def step(c=c, p=p):
                            wait_loads(p)

                            @pl.when(c + 1 < nch)
                            def _():
                                fire(c + 1, 1 - p)

                            pltpu.sync_copy(mhbuf.at[p], acc.at[idx_v.at[p]],
                                            add=True)

                        if coff == 1 and nch % 2 == 1:
                            pl.when(c < nch)(step)
                        else:
                            step()
                    return carry

                lax.fori_loop(0, (nch + 1) // 2, pair, 0)
            return sweep

        @pl.when(cid == 0)
        def _():
            mk_sweep(mh_hbm)()

        @pl.when(cid == 1)
        def _():
            mk_sweep(cm4_hbm)()

        plsc.subcore_barrier()

        def mk_drain(dst_hbm):
            def drain(j, carry):
                b = j * _NS + sid

                @pl.when(b < nblk)
                def _():
                    rs = pl.ds(b * _CG, _CG)
                    pltpu.sync_copy(acc.at[rs], hbb)
                    pltpu.sync_copy(hbb, dst_hbm.at[rs])

                return carry
            return drain

        @pl.when(cid == 0)
        def _():
            lax.fori_loop(0, (nblk + _NS - 1) // _NS, mk_drain(hout_hbm), 0)

        @pl.when(cid == 1)
        def _():
            lax.fori_loop(0, (nblk + _NS - 1) // _NS, mk_drain(cout_hbm), 0)

    return scatter_k


def kernel(node_features, edge_index, edge_attr, coords,
           We1, be1, We2, be2, Wn1, bn1, Wn2, bn2, Wc1, bc1, Wc2, bc2):
    n, d = node_features.shape
    e = edge_index.shape[1]
    ed = edge_attr.shape[1]

    row = edge_index[0]
    col = edge_index[1]
    ct = coords.T
    cx, cy, cz = ct[0], ct[1], ct[2]
    cpad = jnp.pad(coords, ((0, 0), (0, 1)))
    be1r = be1.reshape(1, d)
    be2r = be2.reshape(1, d)
    bn1r = bn1.reshape(1, d)
    bn2r = bn2.reshape(1, d)
    bc1r = bc1.reshape(1, d)
    bc2r = bc2.reshape(1, 1)

    f32 = jnp.float32
    a, b, w2c, bc1e = pl.pallas_call(
        _prep_body,
        out_shape=[jax.ShapeDtypeStruct((n, d), f32),
                   jax.ShapeDtypeStruct((n, d), f32),
                   jax.ShapeDtypeStruct((d, d), f32),
                   jax.ShapeDtypeStruct((1, d), f32)],
    )(node_features, We1, We2, Wc1, bc1r, be2r)

    nsl = 2
    es = e // nsl
    be = 1280
    grid_e = es // be
    lane3 = (grid_e, 1, be)
    r3 = lambda v: v.reshape(lane3)
    full = lambda shape: pl.BlockSpec(shape, lambda i: (0,) * len(shape))
    lane_spec = pl.BlockSpec((1, 1, be), lambda i: (i, 0, 0))
    zh = jnp.zeros((n, d), f32)
    gather_f = _make_gather(n, es, d, cg=128)
    scatter_f = _make_scatter(n, es, d)
    parts = []
    for s in range(nsl):
        row_s = lax.slice_in_dim(row, s * es, (s + 1) * es)
        col_s = lax.slice_in_dim(col, s * es, (s + 1) * es)
        ea_s = lax.slice_in_dim(edge_attr, s * es, (s + 1) * es)
        ar, bc, xr, yr, zr, xc, yc, zc = gather_f(a, b, cx, cy, cz,
                                                  row_s, col_s)
        mh, cm4 = pl.pallas_call(
            _edge_body,
            grid=(grid_e,),
            in_specs=[
                pl.BlockSpec((be, d), lambda i: (i, 0)),
                pl.BlockSpec((be, d), lambda i: (i, 0)),
                pl.BlockSpec((be, ed), lambda i: (i, 0)),
                lane_spec, lane_spec, lane_spec,
                lane_spec, lane_spec, lane_spec,
                full((2 * d + ed, d)),
                full((1, d)),
                full((d, d)),
                full((1, d)),
                full((d, 1)),
                full((1, 1)),
            ],
            out_specs=[
                pl.BlockSpec((be, d), lambda i: (i, 0)),
                pl.BlockSpec((be, d), lambda i: (i, 0)),
            ],
            out_shape=[jax.ShapeDtypeStruct((es, d), f32),
                       jax.ShapeDtypeStruct((es, d), f32)],
        )(ar, bc, ea_s, r3(xr), r3(yr), r3(zr), r3(xc), r3(yc), r3(zc),
          We1, be1r, w2c, bc1e, Wc2, bc2r)
        parts.append(scatter_f(mh, cm4, row_s, zh))

    (hacc0, cacc0), (hacc1, cacc1) = parts

    bn = 1000
    grid_n = n // bn
    nf_new, cnew = pl.pallas_call(
        _post_body,
        grid=(grid_n,),
        in_specs=[
            pl.BlockSpec((bn, d), lambda i: (i, 0)),
            pl.BlockSpec((bn, d), lambda i: (i, 0)),
            pl.BlockSpec((bn, d), lambda i: (i, 0)),
            pl.BlockSpec((bn, d), lambda i: (i, 0)),
            pl.BlockSpec((bn, d), lambda i: (i, 0)),
            pl.BlockSpec((bn, 4), lambda i: (i, 0)),
            full((d, d)),
            full((1, d)),
            full((2 * d, d)),
            full((1, d)),
            full((d, d)),
            full((1, d)),
        ],
        out_specs=[
            pl.BlockSpec((bn, d), lambda i: (i, 0)),
            pl.BlockSpec((bn, 4), lambda i: (i, 0)),
        ],
        out_shape=[jax.ShapeDtypeStruct((n, d), f32),
                   jax.ShapeDtypeStruct((n, 4), f32)],
    )(node_features, hacc0, cacc0, hacc1, cacc1, cpad,
      We2, be2r, Wn1, bn1r, Wn2, bn2r)

    return nf_new, cnew[:, :coords.shape[1]]
